# Initial kernel scaffold; baseline (speedup 1.0000x reference)
import jax
import jax.numpy as jnp
from jax.experimental import pallas as pl


def kernel(x, edge_index, W1_l, b1_l, W1_r, W2_l, b2_l, W2_r):
    raise NotImplementedError("write your pallas kernel here")


# speedup vs baseline: 10.3306x; 10.3306x over previous
import functools

import jax
import jax.numpy as jnp
from jax import lax
from jax.experimental import pallas as pl
from jax.experimental.pallas import tpu as pltpu
from jax.experimental.pallas import tpu_sc as plsc

N = 10000
E = 160000
D_IN = 767
H = 64
D_OUT = 10

NPAD = 10240
KPAD = 768
NC = 2
NS = 16
NW = NC * NS
CH = 128
KCH = 40
EPAD = NW * KCH * CH
RPT = NPAD // NS
MB = 1024
GRID_M = NPAD // MB


def _mm_split_body(x_ref, w_ref, a_ref, b_ref, *, split):
    y = jnp.dot(x_ref[...], w_ref[...], preferred_element_type=jnp.float32)
    a_ref[...] = y[:, :split]
    b_ref[...] = y[:, split:]


def _project_x(x_pad, w1t):
    return pl.pallas_call(
        functools.partial(_mm_split_body, split=H),
        grid=(GRID_M,),
        in_specs=[
            pl.BlockSpec((MB, KPAD), lambda i: (i, 0)),
            pl.BlockSpec((KPAD, 2 * H), lambda i: (0, 0)),
        ],
        out_specs=[
            pl.BlockSpec((MB, H), lambda i: (i, 0)),
            pl.BlockSpec((MB, H), lambda i: (i, 0)),
        ],
        out_shape=[
            jax.ShapeDtypeStruct((NPAD, H), jnp.float32),
            jax.ShapeDtypeStruct((NPAD, H), jnp.float32),
        ],
    )(x_pad, w1t)


def _combine1_body(acc_ref, cnt_ref, r1_ref, b1_ref, w2_ref, p2_ref, r2_ref):
    s = acc_ref[0] + acc_ref[1]
    c = cnt_ref[0] + cnt_ref[1]
    inv = 1.0 / jnp.maximum(c, 1.0)
    h = jnp.maximum(s * inv[:, None] + b1_ref[...] + r1_ref[...], 0.0)
    y = jnp.dot(h, w2_ref[...], preferred_element_type=jnp.float32)
    p2_ref[...] = y[:, :16]
    r2_ref[...] = y[:, 16:]


def _combine1(acc1, cnt, r1, b1r, w2t):
    return pl.pallas_call(
        _combine1_body,
        grid=(GRID_M,),
        in_specs=[
            pl.BlockSpec((NC, MB, H), lambda i: (0, i, 0)),
            pl.BlockSpec((NC, MB), lambda i: (0, i)),
            pl.BlockSpec((MB, H), lambda i: (i, 0)),
            pl.BlockSpec((1, H), lambda i: (0, 0)),
            pl.BlockSpec((H, 32), lambda i: (0, 0)),
        ],
        out_specs=[
            pl.BlockSpec((MB, 16), lambda i: (i, 0)),
            pl.BlockSpec((MB, 16), lambda i: (i, 0)),
        ],
        out_shape=[
            jax.ShapeDtypeStruct((NPAD, 16), jnp.float32),
            jax.ShapeDtypeStruct((NPAD, 16), jnp.float32),
        ],
    )(acc1, cnt, r1, b1r, w2t)


def _combine2_body(acc_ref, cnt_ref, r2_ref, b2_ref, out_ref):
    s = acc_ref[0] + acc_ref[1]
    c = cnt_ref[0] + cnt_ref[1]
    inv = 1.0 / jnp.maximum(c, 1.0)
    out_ref[...] = s * inv[:, None] + b2_ref[...] + r2_ref[...]


def _combine2(acc2, cnt, r2, b2r):
    return pl.pallas_call(
        _combine2_body,
        grid=(GRID_M,),
        in_specs=[
            pl.BlockSpec((NC, MB, 16), lambda i: (0, i, 0)),
            pl.BlockSpec((NC, MB), lambda i: (0, i)),
            pl.BlockSpec((MB, 16), lambda i: (i, 0)),
            pl.BlockSpec((1, 16), lambda i: (0, 0)),
        ],
        out_specs=pl.BlockSpec((MB, 16), lambda i: (i, 0)),
        out_shape=jax.ShapeDtypeStruct((NPAD, 16), jnp.float32),
    )(acc2, cnt, r2, b2r)


def _sc_agg1(p1, src3, dst3, z2d, z1d):
    mesh = plsc.VectorSubcoreMesh(core_axis_name="c", subcore_axis_name="s")

    @functools.partial(
        pl.kernel,
        mesh=mesh,
        compiler_params=pltpu.CompilerParams(use_tc_tiling_on_sc=False),
        out_type=[
            jax.ShapeDtypeStruct((NC, NPAD, H), jnp.float32),
            jax.ShapeDtypeStruct((NC, NPAD), jnp.float32),
        ],
        scratch_types=[
            pltpu.VMEM((KCH, CH), jnp.int32),
            pltpu.VMEM((KCH, CH), jnp.int32),
            pltpu.VMEM((CH, H), jnp.float32),
            pltpu.VMEM((CH,), jnp.float32),
            pltpu.VMEM((RPT, H), jnp.float32),
            pltpu.VMEM((RPT,), jnp.float32),
            pltpu.VMEM_SHARED((NPAD, H), jnp.float32),
            pltpu.VMEM_SHARED((NPAD,), jnp.float32),
            pltpu.SemaphoreType.DMA,
        ],
    )
    def k(p1_hbm, src_hbm, dst_hbm, z2d_hbm, z1d_hbm, acc_out, cnt_out,
          src_v, dst_v, rows_v, ones_v, stg_v, stgc_v, acc_sh, cnt_sh, sem):
        c = lax.axis_index("c")
        s = lax.axis_index("s")
        wid = s * NC + c
        base = s * RPT

        pltpu.sync_copy(src_hbm.at[wid], src_v)
        pltpu.sync_copy(dst_hbm.at[wid], dst_v)
        pltpu.sync_copy(z2d_hbm, stg_v)
        pltpu.sync_copy(z1d_hbm, stgc_v)
        pltpu.sync_copy(stg_v, acc_sh.at[pl.ds(base, RPT)])
        pltpu.sync_copy(stgc_v, cnt_sh.at[pl.ds(base, RPT)])
        for kk in range(CH // 16):
            ones_v[pl.ds(16 * kk, 16)] = jnp.full((16,), 1.0, jnp.float32)
        plsc.subcore_barrier()

        def body(j, carry):
            pltpu.async_copy(p1_hbm.at[src_v.at[j]], rows_v, sem).wait()
            pltpu.sync_copy(rows_v, acc_sh.at[dst_v.at[j]], add=True)
            pltpu.sync_copy(ones_v, cnt_sh.at[dst_v.at[j]], add=True)
            return carry

        lax.fori_loop(0, KCH, body, 0)
        plsc.subcore_barrier()

        pltpu.sync_copy(acc_sh.at[pl.ds(base, RPT)], stg_v)
        pltpu.sync_copy(stg_v, acc_out.at[c].at[pl.ds(base, RPT)])
        pltpu.sync_copy(cnt_sh.at[pl.ds(base, RPT)], stgc_v)
        pltpu.sync_copy(stgc_v, cnt_out.at[c].at[pl.ds(base, RPT)])

    return k(p1, src3, dst3, z2d, z1d)


def _sc_agg2(p2, src3, dst3, z2d):
    mesh = plsc.VectorSubcoreMesh(core_axis_name="c", subcore_axis_name="s")

    @functools.partial(
        pl.kernel,
        mesh=mesh,
        compiler_params=pltpu.CompilerParams(use_tc_tiling_on_sc=False),
        out_type=jax.ShapeDtypeStruct((NC, NPAD, 16), jnp.float32),
        scratch_types=[
            pltpu.VMEM((KCH, CH), jnp.int32),
            pltpu.VMEM((KCH, CH), jnp.int32),
            pltpu.VMEM((CH, 16), jnp.float32),
            pltpu.VMEM((RPT, 16), jnp.float32),
            pltpu.VMEM_SHARED((NPAD, 16), jnp.float32),
            pltpu.SemaphoreType.DMA,
        ],
    )
    def k(p2_hbm, src_hbm, dst_hbm, z2d_hbm, acc_out,
          src_v, dst_v, rows_v, stg_v, acc_sh, sem):
        c = lax.axis_index("c")
        s = lax.axis_index("s")
        wid = s * NC + c
        base = s * RPT

        pltpu.sync_copy(src_hbm.at[wid], src_v)
        pltpu.sync_copy(dst_hbm.at[wid], dst_v)
        pltpu.sync_copy(z2d_hbm, stg_v)
        pltpu.sync_copy(stg_v, acc_sh.at[pl.ds(base, RPT)])
        plsc.subcore_barrier()

        def body(j, carry):
            pltpu.async_copy(p2_hbm.at[src_v.at[j]], rows_v, sem).wait()
            pltpu.sync_copy(rows_v, acc_sh.at[dst_v.at[j]], add=True)
            return carry

        lax.fori_loop(0, KCH, body, 0)
        plsc.subcore_barrier()

        pltpu.sync_copy(acc_sh.at[pl.ds(base, RPT)], stg_v)
        pltpu.sync_copy(stg_v, acc_out.at[c].at[pl.ds(base, RPT)])

    return k(p2, src3, dst3, z2d)


def kernel(x, edge_index, W1_l, b1_l, W1_r, W2_l, b2_l, W2_r):
    f32 = jnp.float32
    x_pad = jnp.zeros((NPAD, KPAD), f32).at[:N, :D_IN].set(x)
    w1t = jnp.zeros((KPAD, 2 * H), f32)
    w1t = w1t.at[:D_IN, :H].set(W1_l.T).at[:D_IN, H:].set(W1_r.T)
    w2t = jnp.zeros((H, 32), f32)
    w2t = w2t.at[:, :D_OUT].set(W2_l.T).at[:, 16:16 + D_OUT].set(W2_r.T)
    b1r = b1_l.reshape(1, H)
    b2r = jnp.zeros((1, 16), f32).at[0, :D_OUT].set(b2_l)

    src = edge_index[0]
    dst = edge_index[1]
    pad_e = EPAD - E
    src3 = jnp.concatenate(
        [src, jnp.zeros((pad_e,), jnp.int32)]).reshape(NW, KCH, CH)
    dst3 = jnp.concatenate(
        [dst, jnp.full((pad_e,), N, jnp.int32)]).reshape(NW, KCH, CH)

    z2d64 = jnp.zeros((RPT, H), f32)
    z2d16 = jnp.zeros((RPT, 16), f32)
    z1d = jnp.zeros((RPT,), f32)

    p1, r1 = _project_x(x_pad, w1t)
    acc1, cnt = _sc_agg1(p1, src3, dst3, z2d64, z1d)
    p2, r2 = _combine1(acc1, cnt, r1, b1r, w2t)
    acc2 = _sc_agg2(p2, src3, dst3, z2d16)
    out16 = _combine2(acc2, cnt, r2, b2r)
    return out16[:N, :D_OUT]


# speedup vs baseline: 11.9101x; 1.1529x over previous
import functools

import jax
import jax.numpy as jnp
from jax import lax
from jax.experimental import pallas as pl
from jax.experimental.pallas import tpu as pltpu
from jax.experimental.pallas import tpu_sc as plsc

N = 10000
E = 160000
D_IN = 767
H = 64
D_OUT = 10

NPAD = 10240
KPAD = 768
NC = 2
NS = 16
NW = NC * NS
CH = 128
KCH = 40
NBUF = 4
EPAD = NW * KCH * CH
RPT = NPAD // NS
MB = 1024
GRID_M = NPAD // MB


def _mm_split_body(x_ref, w_ref, a_ref, b_ref, *, split):
    y = jnp.dot(x_ref[...], w_ref[...], preferred_element_type=jnp.float32)
    a_ref[...] = y[:, :split]
    b_ref[...] = y[:, split:]


def _project_x(x_pad, w1t):
    return pl.pallas_call(
        functools.partial(_mm_split_body, split=H),
        grid=(GRID_M,),
        in_specs=[
            pl.BlockSpec((MB, KPAD), lambda i: (i, 0)),
            pl.BlockSpec((KPAD, 2 * H), lambda i: (0, 0)),
        ],
        out_specs=[
            pl.BlockSpec((MB, H), lambda i: (i, 0)),
            pl.BlockSpec((MB, H), lambda i: (i, 0)),
        ],
        out_shape=[
            jax.ShapeDtypeStruct((NPAD, H), jnp.float32),
            jax.ShapeDtypeStruct((NPAD, H), jnp.float32),
        ],
    )(x_pad, w1t)


def _combine1_body(acc_ref, cnt_ref, r1_ref, b1_ref, w2_ref, p2_ref, r2_ref):
    s = acc_ref[0] + acc_ref[1]
    c = cnt_ref[0] + cnt_ref[1]
    inv = 1.0 / jnp.maximum(c, 1.0)
    h = jnp.maximum(s * inv[:, None] + b1_ref[...] + r1_ref[...], 0.0)
    y = jnp.dot(h, w2_ref[...], preferred_element_type=jnp.float32)
    p2_ref[...] = y[:, :16]
    r2_ref[...] = y[:, 16:]


def _combine1(acc1, cnt, r1, b1r, w2t):
    return pl.pallas_call(
        _combine1_body,
        grid=(GRID_M,),
        in_specs=[
            pl.BlockSpec((NC, MB, H), lambda i: (0, i, 0)),
            pl.BlockSpec((NC, MB), lambda i: (0, i)),
            pl.BlockSpec((MB, H), lambda i: (i, 0)),
            pl.BlockSpec((1, H), lambda i: (0, 0)),
            pl.BlockSpec((H, 32), lambda i: (0, 0)),
        ],
        out_specs=[
            pl.BlockSpec((MB, 16), lambda i: (i, 0)),
            pl.BlockSpec((MB, 16), lambda i: (i, 0)),
        ],
        out_shape=[
            jax.ShapeDtypeStruct((NPAD, 16), jnp.float32),
            jax.ShapeDtypeStruct((NPAD, 16), jnp.float32),
        ],
    )(acc1, cnt, r1, b1r, w2t)


def _combine2_body(acc_ref, cnt_ref, r2_ref, b2_ref, out_ref):
    s = acc_ref[0] + acc_ref[1]
    c = cnt_ref[0] + cnt_ref[1]
    inv = 1.0 / jnp.maximum(c, 1.0)
    out_ref[...] = s * inv[:, None] + b2_ref[...] + r2_ref[...]


def _combine2(acc2, cnt, r2, b2r):
    return pl.pallas_call(
        _combine2_body,
        grid=(GRID_M,),
        in_specs=[
            pl.BlockSpec((NC, MB, 16), lambda i: (0, i, 0)),
            pl.BlockSpec((NC, MB), lambda i: (0, i)),
            pl.BlockSpec((MB, 16), lambda i: (i, 0)),
            pl.BlockSpec((1, 16), lambda i: (0, 0)),
        ],
        out_specs=pl.BlockSpec((MB, 16), lambda i: (i, 0)),
        out_shape=jax.ShapeDtypeStruct((NPAD, 16), jnp.float32),
    )(acc2, cnt, r2, b2r)


def _sc_agg1(p1, src3, dst3, z2d, z1d):
    mesh = plsc.VectorSubcoreMesh(core_axis_name="c", subcore_axis_name="s")

    @functools.partial(
        pl.kernel,
        mesh=mesh,
        compiler_params=pltpu.CompilerParams(use_tc_tiling_on_sc=False),
        out_type=[
            jax.ShapeDtypeStruct((NC, NPAD, H), jnp.float32),
            jax.ShapeDtypeStruct((NC, NPAD), jnp.float32),
        ],
        scratch_types=[
            pltpu.VMEM((KCH, CH), jnp.int32),
            pltpu.VMEM((KCH, CH), jnp.int32),
            pltpu.VMEM((NBUF, CH, H), jnp.float32),
            pltpu.VMEM((CH,), jnp.float32),
            pltpu.VMEM((RPT, H), jnp.float32),
            pltpu.VMEM((RPT,), jnp.float32),
            pltpu.VMEM_SHARED((NPAD, H), jnp.float32),
            pltpu.VMEM_SHARED((NPAD,), jnp.float32),
            [pltpu.SemaphoreType.DMA] * NBUF,
            pltpu.SemaphoreType.DMA,
        ],
    )
    def k(p1_hbm, src_hbm, dst_hbm, z2d_hbm, z1d_hbm, acc_out, cnt_out,
          src_v, dst_v, rows_v, ones_v, stg_v, stgc_v, acc_sh, cnt_sh,
          sems, osem):
        c = lax.axis_index("c")
        s = lax.axis_index("s")
        wid = s * NC + c
        base = s * RPT

        pltpu.sync_copy(src_hbm.at[wid], src_v)
        pltpu.sync_copy(dst_hbm.at[wid], dst_v)
        pltpu.sync_copy(z2d_hbm, stg_v)
        pltpu.sync_copy(z1d_hbm, stgc_v)
        pltpu.sync_copy(stg_v, acc_sh.at[pl.ds(base, RPT)])
        pltpu.sync_copy(stgc_v, cnt_sh.at[pl.ds(base, RPT)])
        for kk in range(CH // 16):
            ones_v[pl.ds(16 * kk, 16)] = jnp.full((16,), 1.0, jnp.float32)
        plsc.subcore_barrier()

        for b in range(NBUF):
            pltpu.async_copy(p1_hbm.at[src_v.at[b]], rows_v.at[b], sems[b])

        def body(g, carry):
            for b in range(NBUF):
                j = g * NBUF + b
                pltpu.make_async_copy(
                    p1_hbm.at[src_v.at[j]], rows_v.at[b], sems[b]).wait()
                pltpu.sync_copy(rows_v.at[b], acc_sh.at[dst_v.at[j]], add=True)
                pltpu.async_copy(
                    ones_v, cnt_sh.at[dst_v.at[j]], osem, add=True)

                @pl.when(j + NBUF < KCH)
                def _():
                    pltpu.async_copy(
                        p1_hbm.at[src_v.at[j + NBUF]], rows_v.at[b], sems[b])
            return carry

        lax.fori_loop(0, KCH // NBUF, body, 0)

        def drain(j, carry):
            pltpu.make_async_copy(
                ones_v, cnt_sh.at[dst_v.at[0]], osem).wait()
            return carry

        lax.fori_loop(0, KCH, drain, 0)
        plsc.subcore_barrier()

        pltpu.sync_copy(acc_sh.at[pl.ds(base, RPT)], stg_v)
        pltpu.sync_copy(stg_v, acc_out.at[c].at[pl.ds(base, RPT)])
        pltpu.sync_copy(cnt_sh.at[pl.ds(base, RPT)], stgc_v)
        pltpu.sync_copy(stgc_v, cnt_out.at[c].at[pl.ds(base, RPT)])

    return k(p1, src3, dst3, z2d, z1d)


def _sc_agg2(p2, src3, dst3, z2d):
    mesh = plsc.VectorSubcoreMesh(core_axis_name="c", subcore_axis_name="s")

    @functools.partial(
        pl.kernel,
        mesh=mesh,
        compiler_params=pltpu.CompilerParams(use_tc_tiling_on_sc=False),
        out_type=jax.ShapeDtypeStruct((NC, NPAD, 16), jnp.float32),
        scratch_types=[
            pltpu.VMEM((KCH, CH), jnp.int32),
            pltpu.VMEM((KCH, CH), jnp.int32),
            pltpu.VMEM((NBUF, CH, 16), jnp.float32),
            pltpu.VMEM((RPT, 16), jnp.float32),
            pltpu.VMEM_SHARED((NPAD, 16), jnp.float32),
            [pltpu.SemaphoreType.DMA] * NBUF,
        ],
    )
    def k(p2_hbm, src_hbm, dst_hbm, z2d_hbm, acc_out,
          src_v, dst_v, rows_v, stg_v, acc_sh, sems):
        c = lax.axis_index("c")
        s = lax.axis_index("s")
        wid = s * NC + c
        base = s * RPT

        pltpu.sync_copy(src_hbm.at[wid], src_v)
        pltpu.sync_copy(dst_hbm.at[wid], dst_v)
        pltpu.sync_copy(z2d_hbm, stg_v)
        pltpu.sync_copy(stg_v, acc_sh.at[pl.ds(base, RPT)])
        plsc.subcore_barrier()

        for b in range(NBUF):
            pltpu.async_copy(p2_hbm.at[src_v.at[b]], rows_v.at[b], sems[b])

        def body(g, carry):
            for b in range(NBUF):
                j = g * NBUF + b
                pltpu.make_async_copy(
                    p2_hbm.at[src_v.at[j]], rows_v.at[b], sems[b]).wait()
                pltpu.sync_copy(rows_v.at[b], acc_sh.at[dst_v.at[j]], add=True)

                @pl.when(j + NBUF < KCH)
                def _():
                    pltpu.async_copy(
                        p2_hbm.at[src_v.at[j + NBUF]], rows_v.at[b], sems[b])
            return carry

        lax.fori_loop(0, KCH // NBUF, body, 0)
        plsc.subcore_barrier()

        pltpu.sync_copy(acc_sh.at[pl.ds(base, RPT)], stg_v)
        pltpu.sync_copy(stg_v, acc_out.at[c].at[pl.ds(base, RPT)])

    return k(p2, src3, dst3, z2d)


def kernel(x, edge_index, W1_l, b1_l, W1_r, W2_l, b2_l, W2_r):
    f32 = jnp.float32
    x_pad = jnp.zeros((NPAD, KPAD), f32).at[:N, :D_IN].set(x)
    w1t = jnp.zeros((KPAD, 2 * H), f32)
    w1t = w1t.at[:D_IN, :H].set(W1_l.T).at[:D_IN, H:].set(W1_r.T)
    w2t = jnp.zeros((H, 32), f32)
    w2t = w2t.at[:, :D_OUT].set(W2_l.T).at[:, 16:16 + D_OUT].set(W2_r.T)
    b1r = b1_l.reshape(1, H)
    b2r = jnp.zeros((1, 16), f32).at[0, :D_OUT].set(b2_l)

    src = edge_index[0]
    dst = edge_index[1]
    pad_e = EPAD - E
    src3 = jnp.concatenate(
        [src, jnp.zeros((pad_e,), jnp.int32)]).reshape(NW, KCH, CH)
    dst3 = jnp.concatenate(
        [dst, jnp.full((pad_e,), N, jnp.int32)]).reshape(NW, KCH, CH)

    z2d64 = jnp.zeros((RPT, H), f32)
    z2d16 = jnp.zeros((RPT, 16), f32)
    z1d = jnp.zeros((RPT,), f32)

    p1, r1 = _project_x(x_pad, w1t)
    acc1, cnt = _sc_agg1(p1, src3, dst3, z2d64, z1d)
    p2, r2 = _combine1(acc1, cnt, r1, b1r, w2t)
    acc2 = _sc_agg2(p2, src3, dst3, z2d16)
    out16 = _combine2(acc2, cnt, r2, b2r)
    return out16[:N, :D_OUT]


# speedup vs baseline: 13.5008x; 1.1336x over previous
import functools

import jax
import jax.numpy as jnp
from jax import lax
from jax.experimental import pallas as pl
from jax.experimental.pallas import tpu as pltpu
from jax.experimental.pallas import tpu_sc as plsc

N = 10000
E = 160000
D_IN = 767
H = 64
D_OUT = 10

NPAD = 10240
NC = 2
NS = 16
NW = NC * NS
CH = 128
KCH = 40
NBUF = 4
EPAD = NW * KCH * CH
RPT = NPAD // NS


def _mm_split_body(x_ref, w_ref, a_ref, b_ref, *, split):
    y = jnp.dot(x_ref[...], w_ref[...], preferred_element_type=jnp.float32)
    a_ref[...] = y[:, :split]
    b_ref[...] = y[:, split:]


def _project_x(x, w1t):
    return pl.pallas_call(
        functools.partial(_mm_split_body, split=H),
        grid=(5,),
        in_specs=[
            pl.BlockSpec((2000, D_IN), lambda i: (i, 0)),
            pl.BlockSpec((D_IN, 2 * H), lambda i: (0, 0)),
        ],
        out_specs=[
            pl.BlockSpec((2000, H), lambda i: (i, 0)),
            pl.BlockSpec((2000, H), lambda i: (i, 0)),
        ],
        out_shape=[
            jax.ShapeDtypeStruct((N, H), jnp.float32),
            jax.ShapeDtypeStruct((N, H), jnp.float32),
        ],
    )(x, w1t)


def _combine1_body(acc_ref, cnt_ref, r1_ref, b1_ref, w2_ref, p2_ref, r2_ref):
    s = acc_ref[0, :N] + acc_ref[1, :N]
    c = cnt_ref[0, :N] + cnt_ref[1, :N]
    inv = 1.0 / jnp.maximum(c, 1.0)
    h = jnp.maximum(s * inv[:, None] + b1_ref[...] + r1_ref[...], 0.0)
    y = jnp.dot(h, w2_ref[...], preferred_element_type=jnp.float32)
    p2_ref[...] = y[:, :16]
    r2_ref[...] = y[:, 16:]


def _combine1(acc1, cnt, r1, b1r, w2t):
    return pl.pallas_call(
        _combine1_body,
        out_shape=[
            jax.ShapeDtypeStruct((N, 16), jnp.float32),
            jax.ShapeDtypeStruct((N, 16), jnp.float32),
        ],
    )(acc1, cnt, r1, b1r, w2t)


def _combine2_body(acc_ref, cnt_ref, r2_ref, b2_ref, out_ref):
    s = acc_ref[0, :N] + acc_ref[1, :N]
    c = cnt_ref[0, :N] + cnt_ref[1, :N]
    inv = 1.0 / jnp.maximum(c, 1.0)
    out_ref[...] = s * inv[:, None] + b2_ref[...] + r2_ref[...]


def _combine2(acc2, cnt, r2, b2r):
    return pl.pallas_call(
        _combine2_body,
        out_shape=jax.ShapeDtypeStruct((N, 16), jnp.float32),
    )(acc2, cnt, r2, b2r)


def _sc_agg1(p1, src3, dst3, z2d, z1d):
    mesh = plsc.VectorSubcoreMesh(core_axis_name="c", subcore_axis_name="s")

    @functools.partial(
        pl.kernel,
        mesh=mesh,
        compiler_params=pltpu.CompilerParams(use_tc_tiling_on_sc=False),
        out_type=[
            jax.ShapeDtypeStruct((NC, NPAD, H), jnp.float32),
            jax.ShapeDtypeStruct((NC, NPAD), jnp.float32),
        ],
        scratch_types=[
            pltpu.VMEM((KCH, CH), jnp.int32),
            pltpu.VMEM((KCH, CH), jnp.int32),
            pltpu.VMEM((NBUF, CH, H), jnp.float32),
            pltpu.VMEM((CH,), jnp.float32),
            pltpu.VMEM((RPT, H), jnp.float32),
            pltpu.VMEM((RPT,), jnp.float32),
            pltpu.VMEM_SHARED((NPAD, H), jnp.float32),
            pltpu.VMEM_SHARED((NPAD,), jnp.float32),
            [pltpu.SemaphoreType.DMA] * NBUF,
            pltpu.SemaphoreType.DMA,
        ],
    )
    def k(p1_hbm, src_hbm, dst_hbm, z2d_hbm, z1d_hbm, acc_out, cnt_out,
          src_v, dst_v, rows_v, ones_v, stg_v, stgc_v, acc_sh, cnt_sh,
          sems, osem):
        c = lax.axis_index("c")
        s = lax.axis_index("s")
        wid = s * NC + c
        base = s * RPT

        pltpu.sync_copy(src_hbm.at[wid], src_v)
        pltpu.sync_copy(dst_hbm.at[wid], dst_v)
        pltpu.sync_copy(z2d_hbm, stg_v)
        pltpu.sync_copy(z1d_hbm, stgc_v)
        pltpu.sync_copy(stg_v, acc_sh.at[pl.ds(base, RPT)])
        pltpu.sync_copy(stgc_v, cnt_sh.at[pl.ds(base, RPT)])
        for kk in range(CH // 16):
            ones_v[pl.ds(16 * kk, 16)] = jnp.full((16,), 1.0, jnp.float32)
        plsc.subcore_barrier()

        for b in range(NBUF):
            pltpu.async_copy(p1_hbm.at[src_v.at[b]], rows_v.at[b], sems[b])

        def body(g, carry):
            for b in range(NBUF):
                j = g * NBUF + b
                pltpu.make_async_copy(
                    p1_hbm.at[src_v.at[j]], rows_v.at[b], sems[b]).wait()
                pltpu.sync_copy(rows_v.at[b], acc_sh.at[dst_v.at[j]], add=True)
                pltpu.async_copy(
                    ones_v, cnt_sh.at[dst_v.at[j]], osem, add=True)

                @pl.when(j + NBUF < KCH)
                def _():
                    pltpu.async_copy(
                        p1_hbm.at[src_v.at[j + NBUF]], rows_v.at[b], sems[b])
            return carry

        lax.fori_loop(0, KCH // NBUF, body, 0)

        def drain(j, carry):
            pltpu.make_async_copy(
                ones_v, cnt_sh.at[dst_v.at[0]], osem).wait()
            return carry

        lax.fori_loop(0, KCH, drain, 0)
        plsc.subcore_barrier()

        pltpu.sync_copy(acc_sh.at[pl.ds(base, RPT)], stg_v)
        pltpu.sync_copy(stg_v, acc_out.at[c].at[pl.ds(base, RPT)])
        pltpu.sync_copy(cnt_sh.at[pl.ds(base, RPT)], stgc_v)
        pltpu.sync_copy(stgc_v, cnt_out.at[c].at[pl.ds(base, RPT)])

    return k(p1, src3, dst3, z2d, z1d)


def _sc_agg2(p2, src3, dst3, z2d):
    mesh = plsc.VectorSubcoreMesh(core_axis_name="c", subcore_axis_name="s")

    @functools.partial(
        pl.kernel,
        mesh=mesh,
        compiler_params=pltpu.CompilerParams(use_tc_tiling_on_sc=False),
        out_type=jax.ShapeDtypeStruct((NC, NPAD, 16), jnp.float32),
        scratch_types=[
            pltpu.VMEM((KCH, CH), jnp.int32),
            pltpu.VMEM((KCH, CH), jnp.int32),
            pltpu.VMEM((NBUF, CH, 16), jnp.float32),
            pltpu.VMEM((RPT, 16), jnp.float32),
            pltpu.VMEM_SHARED((NPAD, 16), jnp.float32),
            [pltpu.SemaphoreType.DMA] * NBUF,
        ],
    )
    def k(p2_hbm, src_hbm, dst_hbm, z2d_hbm, acc_out,
          src_v, dst_v, rows_v, stg_v, acc_sh, sems):
        c = lax.axis_index("c")
        s = lax.axis_index("s")
        wid = s * NC + c
        base = s * RPT

        pltpu.sync_copy(src_hbm.at[wid], src_v)
        pltpu.sync_copy(dst_hbm.at[wid], dst_v)
        pltpu.sync_copy(z2d_hbm, stg_v)
        pltpu.sync_copy(stg_v, acc_sh.at[pl.ds(base, RPT)])
        plsc.subcore_barrier()

        for b in range(NBUF):
            pltpu.async_copy(p2_hbm.at[src_v.at[b]], rows_v.at[b], sems[b])

        def body(g, carry):
            for b in range(NBUF):
                j = g * NBUF + b
                pltpu.make_async_copy(
                    p2_hbm.at[src_v.at[j]], rows_v.at[b], sems[b]).wait()
                pltpu.sync_copy(rows_v.at[b], acc_sh.at[dst_v.at[j]], add=True)

                @pl.when(j + NBUF < KCH)
                def _():
                    pltpu.async_copy(
                        p2_hbm.at[src_v.at[j + NBUF]], rows_v.at[b], sems[b])
            return carry

        lax.fori_loop(0, KCH // NBUF, body, 0)
        plsc.subcore_barrier()

        pltpu.sync_copy(acc_sh.at[pl.ds(base, RPT)], stg_v)
        pltpu.sync_copy(stg_v, acc_out.at[c].at[pl.ds(base, RPT)])

    return k(p2, src3, dst3, z2d)


def kernel(x, edge_index, W1_l, b1_l, W1_r, W2_l, b2_l, W2_r):
    f32 = jnp.float32
    w1t = jnp.concatenate([W1_l, W1_r], axis=0).T
    w2t = jnp.zeros((H, 32), f32)
    w2t = w2t.at[:, :D_OUT].set(W2_l.T).at[:, 16:16 + D_OUT].set(W2_r.T)
    b1r = b1_l.reshape(1, H)
    b2r = jnp.zeros((1, 16), f32).at[0, :D_OUT].set(b2_l)

    src = edge_index[0]
    dst = edge_index[1]
    pad_e = EPAD - E
    pad_dst = N + (jnp.arange(pad_e, dtype=jnp.int32) % (NPAD - N))
    src3 = jnp.concatenate(
        [src, jnp.zeros((pad_e,), jnp.int32)]).reshape(NW, KCH, CH)
    dst3 = jnp.concatenate([dst, pad_dst]).reshape(NW, KCH, CH)

    z2d64 = jnp.zeros((RPT, H), f32)
    z2d16 = jnp.zeros((RPT, 16), f32)
    z1d = jnp.zeros((RPT,), f32)

    p1, r1 = _project_x(x, w1t)
    acc1, cnt = _sc_agg1(p1, src3, dst3, z2d64, z1d)
    p2, r2 = _combine1(acc1, cnt, r1, b1r, w2t)
    acc2 = _sc_agg2(p2, src3, dst3, z2d16)
    out16 = _combine2(acc2, cnt, r2, b2r)
    return out16[:, :D_OUT]


# speedup vs baseline: 13.7250x; 1.0166x over previous
import functools

import jax
import jax.numpy as jnp
from jax import lax
from jax.experimental import pallas as pl
from jax.experimental.pallas import tpu as pltpu
from jax.experimental.pallas import tpu_sc as plsc

N = 10000
E = 160000
D_IN = 767
H = 64
D_OUT = 10

NPAD = 10240
NC = 2
NS = 16
NW = NC * NS
CH = 128
KCH = 40
NBUF = 8
EPAD = NW * KCH * CH
RPT = NPAD // NS


def _mm_split_body(x_ref, w_ref, a_ref, b_ref, *, split):
    y = jnp.dot(x_ref[...], w_ref[...], preferred_element_type=jnp.float32)
    a_ref[...] = y[:, :split]
    b_ref[...] = y[:, split:]


def _project_x(x, w1t):
    return pl.pallas_call(
        functools.partial(_mm_split_body, split=H),
        grid=(5,),
        in_specs=[
            pl.BlockSpec((2000, D_IN), lambda i: (i, 0)),
            pl.BlockSpec((D_IN, 2 * H), lambda i: (0, 0)),
        ],
        out_specs=[
            pl.BlockSpec((2000, H), lambda i: (i, 0)),
            pl.BlockSpec((2000, H), lambda i: (i, 0)),
        ],
        out_shape=[
            jax.ShapeDtypeStruct((N, H), jnp.float32),
            jax.ShapeDtypeStruct((N, H), jnp.float32),
        ],
    )(x, w1t)


def _combine1_body(acc_ref, cnt_ref, r1_ref, b1_ref, w2_ref, p2_ref, r2_ref):
    s = acc_ref[0, :N] + acc_ref[1, :N]
    c = cnt_ref[0, :N] + cnt_ref[1, :N]
    inv = 1.0 / jnp.maximum(c, 1.0)
    h = jnp.maximum(s * inv[:, None] + b1_ref[...] + r1_ref[...], 0.0)
    y = jnp.dot(h, w2_ref[...], preferred_element_type=jnp.float32)
    p2_ref[...] = y[:, :16]
    r2_ref[...] = y[:, 16:]


def _combine1(acc1, cnt, r1, b1r, w2t):
    return pl.pallas_call(
        _combine1_body,
        out_shape=[
            jax.ShapeDtypeStruct((N, 16), jnp.float32),
            jax.ShapeDtypeStruct((N, 16), jnp.float32),
        ],
    )(acc1, cnt, r1, b1r, w2t)


def _combine2_body(acc_ref, cnt_ref, r2_ref, b2_ref, out_ref):
    s = acc_ref[0, :N] + acc_ref[1, :N]
    c = cnt_ref[0, :N] + cnt_ref[1, :N]
    inv = 1.0 / jnp.maximum(c, 1.0)
    out_ref[...] = s * inv[:, None] + b2_ref[...] + r2_ref[...]


def _combine2(acc2, cnt, r2, b2r):
    return pl.pallas_call(
        _combine2_body,
        out_shape=jax.ShapeDtypeStruct((N, 16), jnp.float32),
    )(acc2, cnt, r2, b2r)


def _sc_agg1(p1, src3, dst3, z2d, z1d):
    mesh = plsc.VectorSubcoreMesh(core_axis_name="c", subcore_axis_name="s")

    @functools.partial(
        pl.kernel,
        mesh=mesh,
        compiler_params=pltpu.CompilerParams(use_tc_tiling_on_sc=False),
        out_type=[
            jax.ShapeDtypeStruct((NC, NPAD, H), jnp.float32),
            jax.ShapeDtypeStruct((NC, NPAD), jnp.float32),
        ],
        scratch_types=[
            pltpu.VMEM((KCH, CH), jnp.int32),
            pltpu.VMEM((KCH, CH), jnp.int32),
            pltpu.VMEM((NBUF, CH, H), jnp.float32),
            pltpu.VMEM((CH,), jnp.float32),
            pltpu.VMEM_SHARED((NPAD, H), jnp.float32),
            pltpu.VMEM_SHARED((NPAD,), jnp.float32),
            [pltpu.SemaphoreType.DMA] * NBUF,
            pltpu.SemaphoreType.DMA,
        ],
    )
    def k(p1_hbm, src_hbm, dst_hbm, z2d_hbm, z1d_hbm, acc_out, cnt_out,
          src_v, dst_v, rows_v, ones_v, acc_sh, cnt_sh,
          sems, osem):
        c = lax.axis_index("c")
        s = lax.axis_index("s")
        wid = s * NC + c
        base = s * RPT

        pltpu.sync_copy(src_hbm.at[wid], src_v)
        pltpu.sync_copy(dst_hbm.at[wid], dst_v)
        pltpu.sync_copy(z2d_hbm, acc_sh.at[pl.ds(base, RPT)])
        pltpu.sync_copy(z1d_hbm, cnt_sh.at[pl.ds(base, RPT)])
        for kk in range(CH // 16):
            ones_v[pl.ds(16 * kk, 16)] = jnp.full((16,), 1.0, jnp.float32)
        plsc.subcore_barrier()

        for b in range(NBUF):
            pltpu.async_copy(p1_hbm.at[src_v.at[b]], rows_v.at[b], sems[b])

        def body(g, carry):
            for b in range(NBUF):
                j = g * NBUF + b
                pltpu.make_async_copy(
                    p1_hbm.at[src_v.at[j]], rows_v.at[b], sems[b]).wait()
                pltpu.sync_copy(rows_v.at[b], acc_sh.at[dst_v.at[j]], add=True)
                pltpu.async_copy(
                    ones_v, cnt_sh.at[dst_v.at[j]], osem, add=True)

                @pl.when(j + NBUF < KCH)
                def _():
                    pltpu.async_copy(
                        p1_hbm.at[src_v.at[j + NBUF]], rows_v.at[b], sems[b])
            return carry

        lax.fori_loop(0, KCH // NBUF, body, 0)

        def drain(j, carry):
            pltpu.make_async_copy(
                ones_v, cnt_sh.at[dst_v.at[0]], osem).wait()
            return carry

        lax.fori_loop(0, KCH, drain, 0)
        plsc.subcore_barrier()

        pltpu.sync_copy(acc_sh.at[pl.ds(base, RPT)],
                        acc_out.at[c].at[pl.ds(base, RPT)])
        pltpu.sync_copy(cnt_sh.at[pl.ds(base, RPT)],
                        cnt_out.at[c].at[pl.ds(base, RPT)])

    return k(p1, src3, dst3, z2d, z1d)


def _sc_agg2(p2, src3, dst3, z2d):
    mesh = plsc.VectorSubcoreMesh(core_axis_name="c", subcore_axis_name="s")

    @functools.partial(
        pl.kernel,
        mesh=mesh,
        compiler_params=pltpu.CompilerParams(use_tc_tiling_on_sc=False),
        out_type=jax.ShapeDtypeStruct((NC, NPAD, 16), jnp.float32),
        scratch_types=[
            pltpu.VMEM((KCH, CH), jnp.int32),
            pltpu.VMEM((KCH, CH), jnp.int32),
            pltpu.VMEM((NBUF, CH, 16), jnp.float32),
            pltpu.VMEM_SHARED((NPAD, 16), jnp.float32),
            [pltpu.SemaphoreType.DMA] * NBUF,
        ],
    )
    def k(p2_hbm, src_hbm, dst_hbm, z2d_hbm, acc_out,
          src_v, dst_v, rows_v, acc_sh, sems):
        c = lax.axis_index("c")
        s = lax.axis_index("s")
        wid = s * NC + c
        base = s * RPT

        pltpu.sync_copy(src_hbm.at[wid], src_v)
        pltpu.sync_copy(dst_hbm.at[wid], dst_v)
        pltpu.sync_copy(z2d_hbm, acc_sh.at[pl.ds(base, RPT)])
        plsc.subcore_barrier()

        for b in range(NBUF):
            pltpu.async_copy(p2_hbm.at[src_v.at[b]], rows_v.at[b], sems[b])

        def body(g, carry):
            for b in range(NBUF):
                j = g * NBUF + b
                pltpu.make_async_copy(
                    p2_hbm.at[src_v.at[j]], rows_v.at[b], sems[b]).wait()
                pltpu.sync_copy(rows_v.at[b], acc_sh.at[dst_v.at[j]], add=True)

                @pl.when(j + NBUF < KCH)
                def _():
                    pltpu.async_copy(
                        p2_hbm.at[src_v.at[j + NBUF]], rows_v.at[b], sems[b])
            return carry

        lax.fori_loop(0, KCH // NBUF, body, 0)
        plsc.subcore_barrier()

        pltpu.sync_copy(acc_sh.at[pl.ds(base, RPT)],
                        acc_out.at[c].at[pl.ds(base, RPT)])

    return k(p2, src3, dst3, z2d)


def kernel(x, edge_index, W1_l, b1_l, W1_r, W2_l, b2_l, W2_r):
    f32 = jnp.float32
    w1t = jnp.concatenate([W1_l, W1_r], axis=0).T
    w2t = jnp.zeros((H, 32), f32)
    w2t = w2t.at[:, :D_OUT].set(W2_l.T).at[:, 16:16 + D_OUT].set(W2_r.T)
    b1r = b1_l.reshape(1, H)
    b2r = jnp.zeros((1, 16), f32).at[0, :D_OUT].set(b2_l)

    src = edge_index[0]
    dst = edge_index[1]
    pad_e = EPAD - E
    pad_dst = N + (jnp.arange(pad_e, dtype=jnp.int32) % (NPAD - N))
    src3 = jnp.concatenate(
        [src, jnp.zeros((pad_e,), jnp.int32)]).reshape(NW, KCH, CH)
    dst3 = jnp.concatenate([dst, pad_dst]).reshape(NW, KCH, CH)

    z2d64 = jnp.zeros((RPT, H), f32)
    z2d16 = jnp.zeros((RPT, 16), f32)
    z1d = jnp.zeros((RPT,), f32)

    p1, r1 = _project_x(x, w1t)
    acc1, cnt = _sc_agg1(p1, src3, dst3, z2d64, z1d)
    p2, r2 = _combine1(acc1, cnt, r1, b1r, w2t)
    acc2 = _sc_agg2(p2, src3, dst3, z2d16)
    out16 = _combine2(acc2, cnt, r2, b2r)
    return out16[:, :D_OUT]


# speedup vs baseline: 19.2044x; 1.3992x over previous
import functools

import jax
import jax.numpy as jnp
from jax import lax
from jax.experimental import pallas as pl
from jax.experimental.pallas import tpu as pltpu
from jax.experimental.pallas import tpu_sc as plsc

N = 10000
E = 160000
D_IN = 767
H = 64
D_OUT = 10

NPAD = 10240
NC = 2
NS = 16
NW = NC * NS
CH = 128
KCH = 40
NBUF1 = 4
NBUF2 = 8
TROW = N // NS
EPAD = NW * KCH * CH
RPT = NPAD // NS


def _mm_split_body(x_ref, w_ref, a_ref, b_ref, *, split):
    y = jnp.dot(x_ref[...], w_ref[...], preferred_element_type=jnp.float32)
    a_ref[...] = y[:, :split]
    b_ref[...] = y[:, split:]


def _project_x(x, w1t):
    return pl.pallas_call(
        functools.partial(_mm_split_body, split=H),
        grid=(5,),
        in_specs=[
            pl.BlockSpec((2000, D_IN), lambda i: (i, 0)),
            pl.BlockSpec((D_IN, 2 * H), lambda i: (0, 0)),
        ],
        out_specs=[
            pl.BlockSpec((2000, H), lambda i: (i, 0)),
            pl.BlockSpec((2000, H), lambda i: (i, 0)),
        ],
        out_shape=[
            jax.ShapeDtypeStruct((N, H), jnp.float32),
            jax.ShapeDtypeStruct((N, H), jnp.float32),
        ],
    )(x, w1t)


def _combine1_body(acc_ref, cnt_ref, r1_ref, b1_ref, w2_ref, p2_ref, r2_ref):
    s = acc_ref[0, :N] + acc_ref[1, :N]
    c = cnt_ref[0, :N] + cnt_ref[1, :N]
    inv = 1.0 / jnp.maximum(c, 1.0)
    h = jnp.maximum(s * inv[:, None] + b1_ref[...] + r1_ref[...], 0.0)
    y = jnp.dot(h, w2_ref[...], preferred_element_type=jnp.float32)
    p2_ref[...] = y[:, :16]
    r2_ref[...] = y[:, 16:]


def _combine1(acc1, cnt, r1, b1r, w2t):
    return pl.pallas_call(
        _combine1_body,
        out_shape=[
            jax.ShapeDtypeStruct((N, 16), jnp.float32),
            jax.ShapeDtypeStruct((N, 16), jnp.float32),
        ],
    )(acc1, cnt, r1, b1r, w2t)


def _combine2_body(acc_ref, cnt_ref, r2_ref, b2_ref, out_ref):
    s = acc_ref[0, :N] + acc_ref[1, :N]
    c = cnt_ref[0, :N] + cnt_ref[1, :N]
    inv = 1.0 / jnp.maximum(c, 1.0)
    out_ref[...] = s * inv[:, None] + b2_ref[...] + r2_ref[...]


def _combine2(acc2, cnt, r2, b2r):
    return pl.pallas_call(
        _combine2_body,
        out_shape=jax.ShapeDtypeStruct((N, 16), jnp.float32),
    )(acc2, cnt, r2, b2r)


def _sc_agg1(p1, src3, dst3, z2d, z1d):
    mesh = plsc.VectorSubcoreMesh(core_axis_name="c", subcore_axis_name="s")

    @functools.partial(
        pl.kernel,
        mesh=mesh,
        compiler_params=pltpu.CompilerParams(use_tc_tiling_on_sc=False),
        out_type=[
            jax.ShapeDtypeStruct((NC, NPAD, H), jnp.float32),
            jax.ShapeDtypeStruct((NC, NPAD), jnp.float32),
        ],
        scratch_types=[
            pltpu.VMEM((KCH, CH), jnp.int32),
            pltpu.VMEM((KCH, CH), jnp.int32),
            pltpu.VMEM((NBUF1, CH, H), jnp.float32),
            pltpu.VMEM((CH,), jnp.float32),
            pltpu.VMEM_SHARED((N, H), jnp.float32),
            pltpu.VMEM_SHARED((NPAD, H), jnp.float32),
            pltpu.VMEM_SHARED((NPAD,), jnp.float32),
            [pltpu.SemaphoreType.DMA] * NBUF1,
            pltpu.SemaphoreType.DMA,
        ],
    )
    def k(p1_hbm, src_hbm, dst_hbm, z2d_hbm, z1d_hbm, acc_out, cnt_out,
          src_v, dst_v, rows_v, ones_v, tbl_sh, acc_sh, cnt_sh,
          sems, osem):
        c = lax.axis_index("c")
        s = lax.axis_index("s")
        wid = s * NC + c
        base = s * RPT

        pltpu.sync_copy(src_hbm.at[wid], src_v)
        pltpu.sync_copy(dst_hbm.at[wid], dst_v)
        pltpu.sync_copy(p1_hbm.at[pl.ds(s * TROW, TROW)],
                        tbl_sh.at[pl.ds(s * TROW, TROW)])
        pltpu.sync_copy(z2d_hbm, acc_sh.at[pl.ds(base, RPT)])
        pltpu.sync_copy(z1d_hbm, cnt_sh.at[pl.ds(base, RPT)])
        for kk in range(CH // 16):
            ones_v[pl.ds(16 * kk, 16)] = jnp.full((16,), 1.0, jnp.float32)
        plsc.subcore_barrier()

        for b in range(NBUF1):
            pltpu.async_copy(tbl_sh.at[src_v.at[b]], rows_v.at[b], sems[b])

        def body(g, carry):
            for b in range(NBUF1):
                j = g * NBUF1 + b
                pltpu.make_async_copy(
                    tbl_sh.at[src_v.at[j]], rows_v.at[b], sems[b]).wait()
                pltpu.sync_copy(rows_v.at[b], acc_sh.at[dst_v.at[j]], add=True)
                pltpu.async_copy(
                    ones_v, cnt_sh.at[dst_v.at[j]], osem, add=True)

                @pl.when(j + NBUF1 < KCH)
                def _():
                    pltpu.async_copy(
                        tbl_sh.at[src_v.at[j + NBUF1]], rows_v.at[b], sems[b])
            return carry

        lax.fori_loop(0, KCH // NBUF1, body, 0)

        def drain(j, carry):
            pltpu.make_async_copy(
                ones_v, cnt_sh.at[dst_v.at[0]], osem).wait()
            return carry

        lax.fori_loop(0, KCH, drain, 0)
        plsc.subcore_barrier()

        pltpu.sync_copy(acc_sh.at[pl.ds(base, RPT)],
                        acc_out.at[c].at[pl.ds(base, RPT)])
        pltpu.sync_copy(cnt_sh.at[pl.ds(base, RPT)],
                        cnt_out.at[c].at[pl.ds(base, RPT)])

    return k(p1, src3, dst3, z2d, z1d)


def _sc_agg2(p2, src3, dst3, z2d):
    mesh = plsc.VectorSubcoreMesh(core_axis_name="c", subcore_axis_name="s")

    @functools.partial(
        pl.kernel,
        mesh=mesh,
        compiler_params=pltpu.CompilerParams(use_tc_tiling_on_sc=False),
        out_type=jax.ShapeDtypeStruct((NC, NPAD, 16), jnp.float32),
        scratch_types=[
            pltpu.VMEM((KCH, CH), jnp.int32),
            pltpu.VMEM((KCH, CH), jnp.int32),
            pltpu.VMEM((NBUF2, CH, 16), jnp.float32),
            pltpu.VMEM_SHARED((NPAD, 16), jnp.float32),
            [pltpu.SemaphoreType.DMA] * NBUF2,
        ],
    )
    def k(p2_hbm, src_hbm, dst_hbm, z2d_hbm, acc_out,
          src_v, dst_v, rows_v, acc_sh, sems):
        c = lax.axis_index("c")
        s = lax.axis_index("s")
        wid = s * NC + c
        base = s * RPT

        pltpu.sync_copy(src_hbm.at[wid], src_v)
        pltpu.sync_copy(dst_hbm.at[wid], dst_v)
        pltpu.sync_copy(z2d_hbm, acc_sh.at[pl.ds(base, RPT)])
        plsc.subcore_barrier()

        for b in range(NBUF2):
            pltpu.async_copy(p2_hbm.at[src_v.at[b]], rows_v.at[b], sems[b])

        def body(g, carry):
            for b in range(NBUF2):
                j = g * NBUF2 + b
                pltpu.make_async_copy(
                    p2_hbm.at[src_v.at[j]], rows_v.at[b], sems[b]).wait()
                pltpu.sync_copy(rows_v.at[b], acc_sh.at[dst_v.at[j]], add=True)

                @pl.when(j + NBUF2 < KCH)
                def _():
                    pltpu.async_copy(
                        p2_hbm.at[src_v.at[j + NBUF2]], rows_v.at[b], sems[b])
            return carry

        lax.fori_loop(0, KCH // NBUF2, body, 0)
        plsc.subcore_barrier()

        pltpu.sync_copy(acc_sh.at[pl.ds(base, RPT)],
                        acc_out.at[c].at[pl.ds(base, RPT)])

    return k(p2, src3, dst3, z2d)


def kernel(x, edge_index, W1_l, b1_l, W1_r, W2_l, b2_l, W2_r):
    f32 = jnp.float32
    w1t = jnp.concatenate([W1_l, W1_r], axis=0).T
    w2t = jnp.zeros((H, 32), f32)
    w2t = w2t.at[:, :D_OUT].set(W2_l.T).at[:, 16:16 + D_OUT].set(W2_r.T)
    b1r = b1_l.reshape(1, H)
    b2r = jnp.zeros((1, 16), f32).at[0, :D_OUT].set(b2_l)

    src = edge_index[0]
    dst = edge_index[1]
    pad_e = EPAD - E
    pad_dst = N + (jnp.arange(pad_e, dtype=jnp.int32) % (NPAD - N))
    src3 = jnp.concatenate(
        [src, jnp.zeros((pad_e,), jnp.int32)]).reshape(NW, KCH, CH)
    dst3 = jnp.concatenate([dst, pad_dst]).reshape(NW, KCH, CH)

    z2d64 = jnp.zeros((RPT, H), f32)
    z2d16 = jnp.zeros((RPT, 16), f32)
    z1d = jnp.zeros((RPT,), f32)

    p1, r1 = _project_x(x, w1t)
    acc1, cnt = _sc_agg1(p1, src3, dst3, z2d64, z1d)
    p2, r2 = _combine1(acc1, cnt, r1, b1r, w2t)
    acc2 = _sc_agg2(p2, src3, dst3, z2d16)
    out16 = _combine2(acc2, cnt, r2, b2r)
    return out16[:, :D_OUT]


# speedup vs baseline: 21.3786x; 1.1132x over previous
import functools

import jax
import jax.numpy as jnp
from jax import lax
from jax.experimental import pallas as pl
from jax.experimental.pallas import tpu as pltpu
from jax.experimental.pallas import tpu_sc as plsc

N = 10000
E = 160000
D_IN = 767
H = 64
D_OUT = 10

NPAD = 10240
NC = 2
NS = 16
NW = NC * NS
CH = 128
KCH = 40
NBUF1 = 4
NBUF2 = 8
TROW = N // NS
EPAD = NW * KCH * CH
RPT = NPAD // NS


def _mm_split_body(x_ref, w_ref, a_ref, b_ref, *, split):
    y = jnp.dot(x_ref[...], w_ref[...], preferred_element_type=jnp.float32)
    a_ref[...] = y[:, :split]
    b_ref[...] = y[:, split:]


def _project_x(x, w1t):
    return pl.pallas_call(
        functools.partial(_mm_split_body, split=H),
        grid=(5,),
        in_specs=[
            pl.BlockSpec((2000, D_IN), lambda i: (i, 0)),
            pl.BlockSpec((D_IN, 2 * H), lambda i: (0, 0)),
        ],
        out_specs=[
            pl.BlockSpec((2000, H), lambda i: (i, 0)),
            pl.BlockSpec((2000, H), lambda i: (i, 0)),
        ],
        out_shape=[
            jax.ShapeDtypeStruct((N, H), jnp.float32),
            jax.ShapeDtypeStruct((N, H), jnp.float32),
        ],
    )(x, w1t)


def _combine1_body(acc_ref, cnt_ref, r1_ref, b1_ref, w2_ref, p2_ref, r2_ref):
    s = acc_ref[0, :N] + acc_ref[1, :N]
    c = cnt_ref[0, :N] + cnt_ref[1, :N]
    inv = 1.0 / jnp.maximum(c, 1.0)
    h = jnp.maximum(s * inv[:, None] + b1_ref[...] + r1_ref[...], 0.0)
    y = jnp.dot(h, w2_ref[...], preferred_element_type=jnp.float32)
    p2_ref[...] = y[:, :16]
    r2_ref[...] = y[:, 16:]


def _combine1(acc1, cnt, r1, b1r, w2t):
    return pl.pallas_call(
        _combine1_body,
        out_shape=[
            jax.ShapeDtypeStruct((N, 16), jnp.float32),
            jax.ShapeDtypeStruct((N, 16), jnp.float32),
        ],
    )(acc1, cnt, r1, b1r, w2t)


def _combine2_body(acc_ref, cnt_ref, r2_ref, b2_ref, out_ref):
    s = acc_ref[0, :N] + acc_ref[1, :N]
    c = cnt_ref[0, :N] + cnt_ref[1, :N]
    inv = 1.0 / jnp.maximum(c, 1.0)
    out_ref[...] = s * inv[:, None] + b2_ref[...] + r2_ref[...]


def _combine2(acc2, cnt, r2, b2r):
    return pl.pallas_call(
        _combine2_body,
        out_shape=jax.ShapeDtypeStruct((N, 16), jnp.float32),
    )(acc2, cnt, r2, b2r)


def _sc_agg1(p1, src3, dst3, z2d, z1d):
    mesh = plsc.VectorSubcoreMesh(core_axis_name="c", subcore_axis_name="s")

    @functools.partial(
        pl.kernel,
        mesh=mesh,
        compiler_params=pltpu.CompilerParams(use_tc_tiling_on_sc=False),
        out_type=[
            jax.ShapeDtypeStruct((NC, NPAD, H), jnp.float32),
            jax.ShapeDtypeStruct((NC, NPAD), jnp.float32),
        ],
        scratch_types=[
            pltpu.VMEM((KCH, CH), jnp.int32),
            pltpu.VMEM((KCH, CH), jnp.int32),
            pltpu.VMEM((NBUF1, CH, H), jnp.float32),
            pltpu.VMEM((CH,), jnp.float32),
            pltpu.VMEM_SHARED((N, H), jnp.float32),
            pltpu.VMEM_SHARED((NPAD, H), jnp.float32),
            pltpu.VMEM_SHARED((NPAD,), jnp.float32),
            [pltpu.SemaphoreType.DMA] * NBUF1,
            pltpu.SemaphoreType.DMA,
        ],
    )
    def k(p1_hbm, src_hbm, dst_hbm, z2d_hbm, z1d_hbm, acc_out, cnt_out,
          src_v, dst_v, rows_v, ones_v, tbl_sh, acc_sh, cnt_sh,
          sems, osem):
        c = lax.axis_index("c")
        s = lax.axis_index("s")
        wid = s * NC + c
        base = s * RPT

        pltpu.sync_copy(src_hbm.at[wid], src_v)
        pltpu.sync_copy(dst_hbm.at[wid], dst_v)
        pltpu.sync_copy(p1_hbm.at[pl.ds(s * TROW, TROW)],
                        tbl_sh.at[pl.ds(s * TROW, TROW)])
        pltpu.sync_copy(z2d_hbm, acc_sh.at[pl.ds(base, RPT)])
        pltpu.sync_copy(z1d_hbm, cnt_sh.at[pl.ds(base, RPT)])
        for kk in range(CH // 16):
            ones_v[pl.ds(16 * kk, 16)] = jnp.full((16,), 1.0, jnp.float32)
        plsc.subcore_barrier()

        for b in range(NBUF1):
            pltpu.async_copy(tbl_sh.at[src_v.at[b]], rows_v.at[b], sems[b])

        def body(g, carry):
            for b in range(NBUF1):
                j = g * NBUF1 + b
                pltpu.make_async_copy(
                    tbl_sh.at[src_v.at[j]], rows_v.at[b], sems[b]).wait()
                pltpu.sync_copy(rows_v.at[b], acc_sh.at[dst_v.at[j]], add=True)
                pltpu.async_copy(
                    ones_v, cnt_sh.at[dst_v.at[j]], osem, add=True)

                @pl.when(j + NBUF1 < KCH)
                def _():
                    pltpu.async_copy(
                        tbl_sh.at[src_v.at[j + NBUF1]], rows_v.at[b], sems[b])
            return carry

        lax.fori_loop(0, KCH // NBUF1, body, 0)

        def drain(j, carry):
            pltpu.make_async_copy(
                ones_v, cnt_sh.at[dst_v.at[0]], osem).wait()
            return carry

        lax.fori_loop(0, KCH, drain, 0)
        plsc.subcore_barrier()

        pltpu.sync_copy(acc_sh.at[pl.ds(base, RPT)],
                        acc_out.at[c].at[pl.ds(base, RPT)])
        pltpu.sync_copy(cnt_sh.at[pl.ds(base, RPT)],
                        cnt_out.at[c].at[pl.ds(base, RPT)])

    return k(p1, src3, dst3, z2d, z1d)


def _sc_agg2(p2, src3, dst3, z2d):
    mesh = plsc.VectorSubcoreMesh(core_axis_name="c", subcore_axis_name="s")

    @functools.partial(
        pl.kernel,
        mesh=mesh,
        compiler_params=pltpu.CompilerParams(use_tc_tiling_on_sc=False),
        out_type=jax.ShapeDtypeStruct((NC, NPAD, 16), jnp.float32),
        scratch_types=[
            pltpu.VMEM((KCH, CH), jnp.int32),
            pltpu.VMEM((KCH, CH), jnp.int32),
            pltpu.VMEM((NBUF2, CH, 16), jnp.float32),
            pltpu.VMEM_SHARED((N, 16), jnp.float32),
            pltpu.VMEM_SHARED((NPAD, 16), jnp.float32),
            [pltpu.SemaphoreType.DMA] * NBUF2,
        ],
    )
    def k(p2_hbm, src_hbm, dst_hbm, z2d_hbm, acc_out,
          src_v, dst_v, rows_v, tbl_sh, acc_sh, sems):
        c = lax.axis_index("c")
        s = lax.axis_index("s")
        wid = s * NC + c
        base = s * RPT

        pltpu.sync_copy(src_hbm.at[wid], src_v)
        pltpu.sync_copy(dst_hbm.at[wid], dst_v)
        pltpu.sync_copy(p2_hbm.at[pl.ds(s * TROW, TROW)],
                        tbl_sh.at[pl.ds(s * TROW, TROW)])
        pltpu.sync_copy(z2d_hbm, acc_sh.at[pl.ds(base, RPT)])
        plsc.subcore_barrier()

        for b in range(NBUF2):
            pltpu.async_copy(tbl_sh.at[src_v.at[b]], rows_v.at[b], sems[b])

        def body(g, carry):
            for b in range(NBUF2):
                j = g * NBUF2 + b
                pltpu.make_async_copy(
                    tbl_sh.at[src_v.at[j]], rows_v.at[b], sems[b]).wait()
                pltpu.sync_copy(rows_v.at[b], acc_sh.at[dst_v.at[j]], add=True)

                @pl.when(j + NBUF2 < KCH)
                def _():
                    pltpu.async_copy(
                        tbl_sh.at[src_v.at[j + NBUF2]], rows_v.at[b], sems[b])
            return carry

        lax.fori_loop(0, KCH // NBUF2, body, 0)
        plsc.subcore_barrier()

        pltpu.sync_copy(acc_sh.at[pl.ds(base, RPT)],
                        acc_out.at[c].at[pl.ds(base, RPT)])

    return k(p2, src3, dst3, z2d)


def kernel(x, edge_index, W1_l, b1_l, W1_r, W2_l, b2_l, W2_r):
    f32 = jnp.float32
    w1t = jnp.concatenate([W1_l, W1_r], axis=0).T
    w2t = jnp.zeros((H, 32), f32)
    w2t = w2t.at[:, :D_OUT].set(W2_l.T).at[:, 16:16 + D_OUT].set(W2_r.T)
    b1r = b1_l.reshape(1, H)
    b2r = jnp.zeros((1, 16), f32).at[0, :D_OUT].set(b2_l)

    src = edge_index[0]
    dst = edge_index[1]
    pad_e = EPAD - E
    pad_dst = N + (jnp.arange(pad_e, dtype=jnp.int32) % (NPAD - N))
    src3 = jnp.concatenate(
        [src, jnp.zeros((pad_e,), jnp.int32)]).reshape(NW, KCH, CH)
    dst3 = jnp.concatenate([dst, pad_dst]).reshape(NW, KCH, CH)

    z2d64 = jnp.zeros((RPT, H), f32)
    z2d16 = jnp.zeros((RPT, 16), f32)
    z1d = jnp.zeros((RPT,), f32)

    p1, r1 = _project_x(x, w1t)
    acc1, cnt = _sc_agg1(p1, src3, dst3, z2d64, z1d)
    p2, r2 = _combine1(acc1, cnt, r1, b1r, w2t)
    acc2 = _sc_agg2(p2, src3, dst3, z2d16)
    out16 = _combine2(acc2, cnt, r2, b2r)
    return out16[:, :D_OUT]


# speedup vs baseline: 21.9914x; 1.0287x over previous
import functools

import jax
import jax.numpy as jnp
from jax import lax
from jax.experimental import pallas as pl
from jax.experimental.pallas import tpu as pltpu
from jax.experimental.pallas import tpu_sc as plsc

N = 10000
E = 160000
D_IN = 767
H = 64
D_OUT = 10

NPAD = 10240
NC = 2
NS = 16
NW = NC * NS
CH = 128
KCH = 40
NBUF1 = 4
NBUF2 = 8
TROW = N // NS
EPAD = NW * KCH * CH
RPT = NPAD // NS


def _mm_body(x_ref, w_ref, y_ref):
    y_ref[...] = jnp.dot(x_ref[...], w_ref[...],
                         preferred_element_type=jnp.float32)


def _project_x(x, w1t):
    return pl.pallas_call(
        _mm_body,
        grid=(5,),
        in_specs=[
            pl.BlockSpec((2000, D_IN), lambda i: (i, 0)),
            pl.BlockSpec((D_IN, 2 * H), lambda i: (0, 0)),
        ],
        out_specs=pl.BlockSpec((2000, 2 * H), lambda i: (i, 0)),
        out_shape=jax.ShapeDtypeStruct((N, 2 * H), jnp.float32),
    )(x, w1t)


def _combine1_body(acc_ref, cnt_ref, y1_ref, b1_ref, w2_ref, b2_ref,
                   p2_ref, r2_ref):
    s = acc_ref[0, :N] + acc_ref[1, :N]
    c = cnt_ref[0, :N] + cnt_ref[1, :N]
    inv = 1.0 / jnp.maximum(c, 1.0)
    h = jnp.maximum(s * inv[:, None] + b1_ref[...] + y1_ref[:, H:], 0.0)
    y = jnp.dot(h, w2_ref[...], preferred_element_type=jnp.float32)
    p2_ref[...] = y[:, :16]
    r2_ref[...] = y[:, 16:] + b2_ref[...]


def _combine1(acc1, cnt, y1, b1r, w2t, b2r):
    return pl.pallas_call(
        _combine1_body,
        out_shape=[
            jax.ShapeDtypeStruct((N, 16), jnp.float32),
            jax.ShapeDtypeStruct((N, 16), jnp.float32),
        ],
    )(acc1, cnt, y1, b1r, w2t, b2r)


def _combine2_body(acc_ref, cnt_ref, r2_ref, out_ref):
    s = acc_ref[0, :N] + acc_ref[1, :N]
    c = cnt_ref[0, :N] + cnt_ref[1, :N]
    inv = 1.0 / jnp.maximum(c, 1.0)
    out_ref[...] = s * inv[:, None] + r2_ref[...]


def _combine2(acc2, cnt, r2):
    return pl.pallas_call(
        _combine2_body,
        out_shape=jax.ShapeDtypeStruct((N, 16), jnp.float32),
    )(acc2, cnt, r2)


def _sc_agg1(p1, src3, dst3, z2d, z1d):
    mesh = plsc.VectorSubcoreMesh(core_axis_name="c", subcore_axis_name="s")

    @functools.partial(
        pl.kernel,
        mesh=mesh,
        compiler_params=pltpu.CompilerParams(use_tc_tiling_on_sc=False),
        out_type=[
            jax.ShapeDtypeStruct((NC, NPAD, H), jnp.float32),
            jax.ShapeDtypeStruct((NC, NPAD), jnp.float32),
        ],
        scratch_types=[
            pltpu.VMEM((KCH, CH), jnp.int32),
            pltpu.VMEM((KCH, CH), jnp.int32),
            pltpu.VMEM((NBUF1, CH, H), jnp.float32),
            pltpu.VMEM((CH,), jnp.float32),
            pltpu.VMEM_SHARED((N, H), jnp.float32),
            pltpu.VMEM_SHARED((NPAD, H), jnp.float32),
            pltpu.VMEM_SHARED((NPAD,), jnp.float32),
            [pltpu.SemaphoreType.DMA] * NBUF1,
            pltpu.SemaphoreType.DMA,
        ],
    )
    def k(p1_hbm, src_hbm, dst_hbm, z2d_hbm, z1d_hbm, acc_out, cnt_out,
          src_v, dst_v, rows_v, ones_v, tbl_sh, acc_sh, cnt_sh,
          sems, osem):
        c = lax.axis_index("c")
        s = lax.axis_index("s")
        wid = s * NC + c
        base = s * RPT

        pltpu.sync_copy(src_hbm.at[wid], src_v)
        pltpu.sync_copy(dst_hbm.at[wid], dst_v)
        pltpu.sync_copy(p1_hbm.at[pl.ds(s * TROW, TROW), pl.ds(0, H)],
                        tbl_sh.at[pl.ds(s * TROW, TROW)])
        pltpu.sync_copy(z2d_hbm, acc_sh.at[pl.ds(base, RPT)])
        pltpu.sync_copy(z1d_hbm, cnt_sh.at[pl.ds(base, RPT)])
        for kk in range(CH // 16):
            ones_v[pl.ds(16 * kk, 16)] = jnp.full((16,), 1.0, jnp.float32)
        plsc.subcore_barrier()

        for b in range(NBUF1):
            pltpu.async_copy(tbl_sh.at[src_v.at[b]], rows_v.at[b], sems[b])

        def body(g, carry):
            for b in range(NBUF1):
                j = g * NBUF1 + b
                pltpu.make_async_copy(
                    tbl_sh.at[src_v.at[j]], rows_v.at[b], sems[b]).wait()
                pltpu.sync_copy(rows_v.at[b], acc_sh.at[dst_v.at[j]], add=True)
                pltpu.async_copy(
                    ones_v, cnt_sh.at[dst_v.at[j]], osem, add=True)

                @pl.when(j + NBUF1 < KCH)
                def _():
                    pltpu.async_copy(
                        tbl_sh.at[src_v.at[j + NBUF1]], rows_v.at[b], sems[b])
            return carry

        lax.fori_loop(0, KCH // NBUF1, body, 0)

        def drain(j, carry):
            pltpu.make_async_copy(
                ones_v, cnt_sh.at[dst_v.at[0]], osem).wait()
            return carry

        lax.fori_loop(0, KCH, drain, 0)
        plsc.subcore_barrier()

        pltpu.sync_copy(acc_sh.at[pl.ds(base, RPT)],
                        acc_out.at[c].at[pl.ds(base, RPT)])
        pltpu.sync_copy(cnt_sh.at[pl.ds(base, RPT)],
                        cnt_out.at[c].at[pl.ds(base, RPT)])

    return k(p1, src3, dst3, z2d, z1d)


def _sc_agg2(p2, src3, dst3, z2d):
    mesh = plsc.VectorSubcoreMesh(core_axis_name="c", subcore_axis_name="s")

    @functools.partial(
        pl.kernel,
        mesh=mesh,
        compiler_params=pltpu.CompilerParams(use_tc_tiling_on_sc=False),
        out_type=jax.ShapeDtypeStruct((NC, NPAD, 16), jnp.float32),
        scratch_types=[
            pltpu.VMEM((KCH, CH), jnp.int32),
            pltpu.VMEM((KCH, CH), jnp.int32),
            pltpu.VMEM((NBUF2, CH, 16), jnp.float32),
            pltpu.VMEM_SHARED((N, 16), jnp.float32),
            pltpu.VMEM_SHARED((NPAD, 16), jnp.float32),
            [pltpu.SemaphoreType.DMA] * NBUF2,
        ],
    )
    def k(p2_hbm, src_hbm, dst_hbm, z2d_hbm, acc_out,
          src_v, dst_v, rows_v, tbl_sh, acc_sh, sems):
        c = lax.axis_index("c")
        s = lax.axis_index("s")
        wid = s * NC + c
        base = s * RPT

        pltpu.sync_copy(src_hbm.at[wid], src_v)
        pltpu.sync_copy(dst_hbm.at[wid], dst_v)
        pltpu.sync_copy(p2_hbm.at[pl.ds(s * TROW, TROW)],
                        tbl_sh.at[pl.ds(s * TROW, TROW)])
        pltpu.sync_copy(z2d_hbm, acc_sh.at[pl.ds(base, RPT)])
        plsc.subcore_barrier()

        for b in range(NBUF2):
            pltpu.async_copy(tbl_sh.at[src_v.at[b]], rows_v.at[b], sems[b])

        def body(g, carry):
            for b in range(NBUF2):
                j = g * NBUF2 + b
                pltpu.make_async_copy(
                    tbl_sh.at[src_v.at[j]], rows_v.at[b], sems[b]).wait()
                pltpu.sync_copy(rows_v.at[b], acc_sh.at[dst_v.at[j]], add=True)

                @pl.when(j + NBUF2 < KCH)
                def _():
                    pltpu.async_copy(
                        tbl_sh.at[src_v.at[j + NBUF2]], rows_v.at[b], sems[b])
            return carry

        lax.fori_loop(0, KCH // NBUF2, body, 0)
        plsc.subcore_barrier()

        pltpu.sync_copy(acc_sh.at[pl.ds(base, RPT)],
                        acc_out.at[c].at[pl.ds(base, RPT)])

    return k(p2, src3, dst3, z2d)


def kernel(x, edge_index, W1_l, b1_l, W1_r, W2_l, b2_l, W2_r):
    f32 = jnp.float32
    w1t = jnp.concatenate([W1_l, W1_r], axis=0).T
    w2t = jnp.zeros((H, 32), f32)
    w2t = w2t.at[:, :D_OUT].set(W2_l.T).at[:, 16:16 + D_OUT].set(W2_r.T)
    b1r = b1_l.reshape(1, H)
    b2r = jnp.zeros((1, 16), f32).at[0, :D_OUT].set(b2_l)

    src = edge_index[0]
    dst = edge_index[1]
    pad_e = EPAD - E
    pad_dst = N + (jnp.arange(pad_e, dtype=jnp.int32) % (NPAD - N))
    src3 = jnp.concatenate(
        [src, jnp.zeros((pad_e,), jnp.int32)]).reshape(NW, KCH, CH)
    dst3 = jnp.concatenate([dst, pad_dst]).reshape(NW, KCH, CH)

    z2d64 = jnp.zeros((RPT, H), f32)
    z2d16 = jnp.zeros((RPT, 16), f32)
    z1d = jnp.zeros((RPT,), f32)

    y1 = _project_x(x, w1t)
    acc1, cnt = _sc_agg1(y1, src3, dst3, z2d64, z1d)
    p2, r2 = _combine1(acc1, cnt, y1, b1r, w2t, b2r)
    acc2 = _sc_agg2(p2, src3, dst3, z2d16)
    out16 = _combine2(acc2, cnt, r2)
    return out16[:, :D_OUT]


# speedup vs baseline: 23.6302x; 1.0745x over previous
import functools

import jax
import jax.numpy as jnp
from jax import lax
from jax.experimental import pallas as pl
from jax.experimental.pallas import tpu as pltpu
from jax.experimental.pallas import tpu_sc as plsc

N = 10000
E = 160000
D_IN = 767
H = 64
D_OUT = 10

NPAD = 10240
NC = 2
NS = 16
NW = NC * NS
CH = 128
KCH = 40
NBUF1 = 4
NBUF2 = 8
TROW = N // NS
EPAD = NW * KCH * CH
RPT = NPAD // NS


def _mm_body(x_ref, w_ref, y_ref):
    y_ref[...] = jnp.dot(x_ref[...], w_ref[...],
                         preferred_element_type=jnp.float32)


def _project_x(x, w1t):
    return pl.pallas_call(
        _mm_body,
        grid=(5,),
        in_specs=[
            pl.BlockSpec((2000, D_IN), lambda i: (i, 0)),
            pl.BlockSpec((D_IN, 2 * H), lambda i: (0, 0)),
        ],
        out_specs=pl.BlockSpec((2000, 2 * H), lambda i: (i, 0)),
        out_shape=jax.ShapeDtypeStruct((N, 2 * H), jnp.float32),
    )(x, w1t)


def _combine1_body(acc_ref, cnt_ref, y1_ref, b1_ref, w2_ref, b2_ref, y2_ref):
    s = acc_ref[:N, :H] + acc_ref[:N, H:]
    c = cnt_ref[0, :N] + cnt_ref[1, :N]
    inv = 1.0 / jnp.maximum(c, 1.0)
    h = jnp.maximum(s * inv[:, None] + b1_ref[...] + y1_ref[:, H:], 0.0)
    y2_ref[...] = jnp.dot(h, w2_ref[...],
                          preferred_element_type=jnp.float32) + b2_ref[...]


def _combine1(acc1, cnt, y1, b1r, w2t, b2r):
    return pl.pallas_call(
        _combine1_body,
        out_shape=jax.ShapeDtypeStruct((N, 128), jnp.float32),
    )(acc1, cnt, y1, b1r, w2t, b2r)


def _combine2_body(acc_ref, cnt_ref, y2_ref, out_ref):
    s = acc_ref[0, :N] + acc_ref[1, :N]
    c = cnt_ref[0, :N] + cnt_ref[1, :N]
    inv = 1.0 / jnp.maximum(c, 1.0)
    out = s * inv[:, None] + y2_ref[:, 16:32]
    out_ref[...] = out[:, :D_OUT]


def _combine2(acc2, cnt, y2):
    return pl.pallas_call(
        _combine2_body,
        out_shape=jax.ShapeDtypeStruct((N, D_OUT), jnp.float32),
    )(acc2, cnt, y2)


def _sc_agg1(p1, src3, dst3, z2d, z1d):
    mesh = plsc.VectorSubcoreMesh(core_axis_name="c", subcore_axis_name="s")

    @functools.partial(
        pl.kernel,
        mesh=mesh,
        compiler_params=pltpu.CompilerParams(use_tc_tiling_on_sc=False),
        out_type=[
            jax.ShapeDtypeStruct((NPAD, 2 * H), jnp.float32),
            jax.ShapeDtypeStruct((NC, NPAD), jnp.float32),
        ],
        scratch_types=[
            pltpu.VMEM((KCH, CH), jnp.int32),
            pltpu.VMEM((KCH, CH), jnp.int32),
            pltpu.VMEM((NBUF1, CH, H), jnp.float32),
            pltpu.VMEM((CH,), jnp.float32),
            pltpu.VMEM_SHARED((N, H), jnp.float32),
            pltpu.VMEM_SHARED((NPAD, H), jnp.float32),
            pltpu.VMEM_SHARED((NPAD,), jnp.float32),
            [pltpu.SemaphoreType.DMA] * NBUF1,
            pltpu.SemaphoreType.DMA,
        ],
    )
    def k(p1_hbm, src_hbm, dst_hbm, z2d_hbm, z1d_hbm, acc_out, cnt_out,
          src_v, dst_v, rows_v, ones_v, tbl_sh, acc_sh, cnt_sh,
          sems, osem):
        c = lax.axis_index("c")
        s = lax.axis_index("s")
        wid = s * NC + c
        base = s * RPT

        pltpu.sync_copy(src_hbm.at[wid], src_v)
        pltpu.sync_copy(dst_hbm.at[wid], dst_v)
        pltpu.sync_copy(p1_hbm.at[pl.ds(s * TROW, TROW), pl.ds(0, H)],
                        tbl_sh.at[pl.ds(s * TROW, TROW)])
        pltpu.sync_copy(z2d_hbm, acc_sh.at[pl.ds(base, RPT)])
        pltpu.sync_copy(z1d_hbm, cnt_sh.at[pl.ds(base, RPT)])
        for kk in range(CH // 16):
            ones_v[pl.ds(16 * kk, 16)] = jnp.full((16,), 1.0, jnp.float32)
        plsc.subcore_barrier()

        for b in range(NBUF1):
            pltpu.async_copy(tbl_sh.at[src_v.at[b]], rows_v.at[b], sems[b])

        def body(g, carry):
            for b in range(NBUF1):
                j = g * NBUF1 + b
                pltpu.make_async_copy(
                    tbl_sh.at[src_v.at[j]], rows_v.at[b], sems[b]).wait()
                pltpu.sync_copy(rows_v.at[b], acc_sh.at[dst_v.at[j]], add=True)
                pltpu.async_copy(
                    ones_v, cnt_sh.at[dst_v.at[j]], osem, add=True)

                @pl.when(j + NBUF1 < KCH)
                def _():
                    pltpu.async_copy(
                        tbl_sh.at[src_v.at[j + NBUF1]], rows_v.at[b], sems[b])
            return carry

        lax.fori_loop(0, KCH // NBUF1, body, 0)

        def drain(j, carry):
            pltpu.make_async_copy(
                ones_v, cnt_sh.at[dst_v.at[0]], osem).wait()
            return carry

        lax.fori_loop(0, KCH, drain, 0)
        plsc.subcore_barrier()

        pltpu.sync_copy(
            acc_sh.at[pl.ds(base, RPT)],
            acc_out.at[pl.ds(base, RPT), pl.ds(c * H, H)])
        pltpu.sync_copy(cnt_sh.at[pl.ds(base, RPT)],
                        cnt_out.at[c].at[pl.ds(base, RPT)])

    return k(p1, src3, dst3, z2d, z1d)


def _sc_agg2(p2, src3, dst3, z2d):
    mesh = plsc.VectorSubcoreMesh(core_axis_name="c", subcore_axis_name="s")

    @functools.partial(
        pl.kernel,
        mesh=mesh,
        compiler_params=pltpu.CompilerParams(use_tc_tiling_on_sc=False),
        out_type=jax.ShapeDtypeStruct((NC, NPAD, 16), jnp.float32),
        scratch_types=[
            pltpu.VMEM((KCH, CH), jnp.int32),
            pltpu.VMEM((KCH, CH), jnp.int32),
            pltpu.VMEM((NBUF2, CH, 16), jnp.float32),
            pltpu.VMEM_SHARED((N, 16), jnp.float32),
            pltpu.VMEM_SHARED((NPAD, 16), jnp.float32),
            [pltpu.SemaphoreType.DMA] * NBUF2,
        ],
    )
    def k(p2_hbm, src_hbm, dst_hbm, z2d_hbm, acc_out,
          src_v, dst_v, rows_v, tbl_sh, acc_sh, sems):
        c = lax.axis_index("c")
        s = lax.axis_index("s")
        wid = s * NC + c
        base = s * RPT

        pltpu.sync_copy(src_hbm.at[wid], src_v)
        pltpu.sync_copy(dst_hbm.at[wid], dst_v)
        pltpu.sync_copy(
            p2_hbm.at[pl.ds(s * TROW, TROW), pl.ds(0, 16)],
            tbl_sh.at[pl.ds(s * TROW, TROW)])
        pltpu.sync_copy(z2d_hbm, acc_sh.at[pl.ds(base, RPT)])
        plsc.subcore_barrier()

        for b in range(NBUF2):
            pltpu.async_copy(tbl_sh.at[src_v.at[b]], rows_v.at[b], sems[b])

        def body(g, carry):
            for b in range(NBUF2):
                j = g * NBUF2 + b
                pltpu.make_async_copy(
                    tbl_sh.at[src_v.at[j]], rows_v.at[b], sems[b]).wait()
                pltpu.sync_copy(rows_v.at[b], acc_sh.at[dst_v.at[j]], add=True)

                @pl.when(j + NBUF2 < KCH)
                def _():
                    pltpu.async_copy(
                        tbl_sh.at[src_v.at[j + NBUF2]], rows_v.at[b], sems[b])
            return carry

        lax.fori_loop(0, KCH // NBUF2, body, 0)
        plsc.subcore_barrier()

        pltpu.sync_copy(acc_sh.at[pl.ds(base, RPT)],
                        acc_out.at[c].at[pl.ds(base, RPT)])

    return k(p2, src3, dst3, z2d)


def kernel(x, edge_index, W1_l, b1_l, W1_r, W2_l, b2_l, W2_r):
    f32 = jnp.float32
    w1t = jnp.concatenate([W1_l, W1_r], axis=0).T
    w2t = jnp.zeros((H, 128), f32)
    w2t = w2t.at[:, :D_OUT].set(W2_l.T).at[:, 16:16 + D_OUT].set(W2_r.T)
    b1r = b1_l.reshape(1, H)
    b2r = jnp.zeros((1, 128), f32).at[0, 16:16 + D_OUT].set(b2_l)

    src = edge_index[0]
    dst = edge_index[1]
    pad_e = EPAD - E
    pad_dst = N + (jnp.arange(pad_e, dtype=jnp.int32) % (NPAD - N))
    src3 = jnp.concatenate(
        [src, jnp.zeros((pad_e,), jnp.int32)]).reshape(NW, KCH, CH)
    dst3 = jnp.concatenate([dst, pad_dst]).reshape(NW, KCH, CH)

    z2d64 = jnp.zeros((RPT, H), f32)
    z2d16 = jnp.zeros((RPT, 16), f32)
    z1d = jnp.zeros((RPT,), f32)

    y1 = _project_x(x, w1t)
    acc1, cnt = _sc_agg1(y1, src3, dst3, z2d64, z1d)
    y2 = _combine1(acc1, cnt, y1, b1r, w2t, b2r)
    acc2 = _sc_agg2(y2, src3, dst3, z2d16)
    return _combine2(acc2, cnt, y2)


# speedup vs baseline: 24.4812x; 1.0360x over previous
import functools

import jax
import jax.numpy as jnp
from jax import lax
from jax.experimental import pallas as pl
from jax.experimental.pallas import tpu as pltpu
from jax.experimental.pallas import tpu_sc as plsc

N = 10000
E = 160000
D_IN = 767
H = 64
D_OUT = 10

NPAD = 10240
NC = 2
NS = 16
NW = NC * NS
CH = 128
KCH = 40
NBUF1 = 4
NBUF2 = 8
TROW = N // NS
EPAD = NW * KCH * CH
RPT = NPAD // NS


def _mm_body(x_ref, w_ref, y_ref):
    y_ref[...] = jnp.dot(x_ref[...], w_ref[...],
                         preferred_element_type=jnp.float32)


def _project_x(x, w1t):
    return pl.pallas_call(
        _mm_body,
        grid=(5,),
        in_specs=[
            pl.BlockSpec((2000, D_IN), lambda i: (i, 0)),
            pl.BlockSpec((D_IN, 2 * H), lambda i: (0, 0)),
        ],
        out_specs=pl.BlockSpec((2000, 2 * H), lambda i: (i, 0)),
        out_shape=jax.ShapeDtypeStruct((N, 2 * H), jnp.float32),
    )(x, w1t)


def _combine1_body(acc_ref, cnt_ref, y1_ref, b1_ref, w2_ref, b2_ref, y2_ref):
    s = acc_ref[:N, :H] + acc_ref[:N, H:]
    c = cnt_ref[0, :N] + cnt_ref[1, :N]
    inv = 1.0 / jnp.maximum(c, 1.0)
    h = jnp.maximum(s * inv[:, None] + b1_ref[...] + y1_ref[:, H:], 0.0)
    y2_ref[...] = jnp.dot(h, w2_ref[...],
                          preferred_element_type=jnp.float32) + b2_ref[...]


def _combine1(acc1, cnt, y1, b1r, w2t, b2r):
    return pl.pallas_call(
        _combine1_body,
        out_shape=jax.ShapeDtypeStruct((N, 128), jnp.float32),
    )(acc1, cnt, y1, b1r, w2t, b2r)


def _combine2_body(acc_ref, cnt_ref, y2_ref, out_ref):
    s = acc_ref[0, :N] + acc_ref[1, :N]
    c = cnt_ref[0, :N] + cnt_ref[1, :N]
    inv = 1.0 / jnp.maximum(c, 1.0)
    out = s * inv[:, None] + y2_ref[:, 16:32]
    out_ref[...] = out[:, :D_OUT]


def _combine2(acc2, cnt, y2):
    return pl.pallas_call(
        _combine2_body,
        out_shape=jax.ShapeDtypeStruct((N, D_OUT), jnp.float32),
    )(acc2, cnt, y2)


def _sc_agg1(p1, src3, dst3, z2d, z1d):
    mesh = plsc.VectorSubcoreMesh(core_axis_name="c", subcore_axis_name="s")

    @functools.partial(
        pl.kernel,
        mesh=mesh,
        compiler_params=pltpu.CompilerParams(use_tc_tiling_on_sc=False),
        out_type=[
            jax.ShapeDtypeStruct((NPAD, 2 * H), jnp.float32),
            jax.ShapeDtypeStruct((NC, NPAD), jnp.float32),
        ],
        scratch_types=[
            pltpu.VMEM((KCH, CH), jnp.int32),
            pltpu.VMEM((KCH, CH), jnp.int32),
            pltpu.VMEM((NBUF1, CH, H), jnp.float32),
            pltpu.VMEM((CH,), jnp.float32),
            pltpu.VMEM_SHARED((N, H), jnp.float32),
            pltpu.VMEM_SHARED((NPAD, H), jnp.float32),
            pltpu.VMEM_SHARED((NPAD,), jnp.float32),
            [pltpu.SemaphoreType.DMA] * NBUF1,
            [pltpu.SemaphoreType.DMA] * NBUF1,
            pltpu.SemaphoreType.DMA,
        ],
    )
    def k(p1_hbm, src_hbm, dst_hbm, z2d_hbm, z1d_hbm, acc_out, cnt_out,
          src_v, dst_v, rows_v, ones_v, tbl_sh, acc_sh, cnt_sh,
          sems, ssems, osem):
        c = lax.axis_index("c")
        s = lax.axis_index("s")
        wid = s * NC + c
        base = s * RPT

        pltpu.sync_copy(src_hbm.at[wid], src_v)
        pltpu.sync_copy(dst_hbm.at[wid], dst_v)
        pltpu.sync_copy(p1_hbm.at[pl.ds(s * TROW, TROW), pl.ds(0, H)],
                        tbl_sh.at[pl.ds(s * TROW, TROW)])
        pltpu.sync_copy(z2d_hbm, acc_sh.at[pl.ds(base, RPT)])
        pltpu.sync_copy(z1d_hbm, cnt_sh.at[pl.ds(base, RPT)])
        for kk in range(CH // 16):
            ones_v[pl.ds(16 * kk, 16)] = jnp.full((16,), 1.0, jnp.float32)
        plsc.subcore_barrier()

        for b in range(NBUF1 - 1):
            pltpu.async_copy(tbl_sh.at[src_v.at[b]], rows_v.at[b], sems[b])

        def body(g, carry):
            for b in range(NBUF1):
                j = g * NBUF1 + b
                pltpu.make_async_copy(
                    tbl_sh.at[src_v.at[j]], rows_v.at[b], sems[b]).wait()
                pltpu.async_copy(
                    rows_v.at[b], acc_sh.at[dst_v.at[j]], ssems[b], add=True)
                pltpu.async_copy(
                    ones_v, cnt_sh.at[dst_v.at[j]], osem, add=True)

                bn = (b + NBUF1 - 1) % NBUF1

                @pl.when(j + NBUF1 - 1 < KCH)
                def _():
                    @pl.when(j >= 1)
                    def _():
                        pltpu.make_async_copy(
                            rows_v.at[bn], acc_sh.at[dst_v.at[0]],
                            ssems[bn]).wait()

                    pltpu.async_copy(
                        tbl_sh.at[src_v.at[j + NBUF1 - 1]], rows_v.at[bn],
                        sems[bn])
            return carry

        lax.fori_loop(0, KCH // NBUF1, body, 0)

        for b in range(NBUF1):
            pltpu.make_async_copy(
                rows_v.at[b], acc_sh.at[dst_v.at[0]], ssems[b]).wait()

        def drain(j, carry):
            pltpu.make_async_copy(
                ones_v, cnt_sh.at[dst_v.at[0]], osem).wait()
            return carry

        lax.fori_loop(0, KCH, drain, 0)
        plsc.subcore_barrier()

        pltpu.sync_copy(
            acc_sh.at[pl.ds(base, RPT)],
            acc_out.at[pl.ds(base, RPT), pl.ds(c * H, H)])
        pltpu.sync_copy(cnt_sh.at[pl.ds(base, RPT)],
                        cnt_out.at[c].at[pl.ds(base, RPT)])

    return k(p1, src3, dst3, z2d, z1d)


def _sc_agg2(p2, src3, dst3, z2d):
    mesh = plsc.VectorSubcoreMesh(core_axis_name="c", subcore_axis_name="s")

    @functools.partial(
        pl.kernel,
        mesh=mesh,
        compiler_params=pltpu.CompilerParams(use_tc_tiling_on_sc=False),
        out_type=jax.ShapeDtypeStruct((NC, NPAD, 16), jnp.float32),
        scratch_types=[
            pltpu.VMEM((KCH, CH), jnp.int32),
            pltpu.VMEM((KCH, CH), jnp.int32),
            pltpu.VMEM((NBUF2, CH, 16), jnp.float32),
            pltpu.VMEM_SHARED((N, 16), jnp.float32),
            pltpu.VMEM_SHARED((NPAD, 16), jnp.float32),
            [pltpu.SemaphoreType.DMA] * NBUF2,
            [pltpu.SemaphoreType.DMA] * NBUF2,
        ],
    )
    def k(p2_hbm, src_hbm, dst_hbm, z2d_hbm, acc_out,
          src_v, dst_v, rows_v, tbl_sh, acc_sh, sems, ssems):
        c = lax.axis_index("c")
        s = lax.axis_index("s")
        wid = s * NC + c
        base = s * RPT

        pltpu.sync_copy(src_hbm.at[wid], src_v)
        pltpu.sync_copy(dst_hbm.at[wid], dst_v)
        pltpu.sync_copy(
            p2_hbm.at[pl.ds(s * TROW, TROW), pl.ds(0, 16)],
            tbl_sh.at[pl.ds(s * TROW, TROW)])
        pltpu.sync_copy(z2d_hbm, acc_sh.at[pl.ds(base, RPT)])
        plsc.subcore_barrier()

        for b in range(NBUF2 - 1):
            pltpu.async_copy(tbl_sh.at[src_v.at[b]], rows_v.at[b], sems[b])

        def body(g, carry):
            for b in range(NBUF2):
                j = g * NBUF2 + b
                pltpu.make_async_copy(
                    tbl_sh.at[src_v.at[j]], rows_v.at[b], sems[b]).wait()
                pltpu.async_copy(
                    rows_v.at[b], acc_sh.at[dst_v.at[j]], ssems[b], add=True)

                bn = (b + NBUF2 - 1) % NBUF2

                @pl.when(j + NBUF2 - 1 < KCH)
                def _():
                    @pl.when(j >= 1)
                    def _():
                        pltpu.make_async_copy(
                            rows_v.at[bn], acc_sh.at[dst_v.at[0]],
                            ssems[bn]).wait()

                    pltpu.async_copy(
                        tbl_sh.at[src_v.at[j + NBUF2 - 1]], rows_v.at[bn],
                        sems[bn])
            return carry

        lax.fori_loop(0, KCH // NBUF2, body, 0)

        for b in range(NBUF2):
            pltpu.make_async_copy(
                rows_v.at[b], acc_sh.at[dst_v.at[0]], ssems[b]).wait()

        plsc.subcore_barrier()

        pltpu.sync_copy(acc_sh.at[pl.ds(base, RPT)],
                        acc_out.at[c].at[pl.ds(base, RPT)])

    return k(p2, src3, dst3, z2d)


def kernel(x, edge_index, W1_l, b1_l, W1_r, W2_l, b2_l, W2_r):
    f32 = jnp.float32
    w1t = jnp.concatenate([W1_l, W1_r], axis=0).T
    w2t = jnp.zeros((H, 128), f32)
    w2t = w2t.at[:, :D_OUT].set(W2_l.T).at[:, 16:16 + D_OUT].set(W2_r.T)
    b1r = b1_l.reshape(1, H)
    b2r = jnp.zeros((1, 128), f32).at[0, 16:16 + D_OUT].set(b2_l)

    src = edge_index[0]
    dst = edge_index[1]
    pad_e = EPAD - E
    pad_dst = N + (jnp.arange(pad_e, dtype=jnp.int32) % (NPAD - N))
    src3 = jnp.concatenate(
        [src, jnp.zeros((pad_e,), jnp.int32)]).reshape(NW, KCH, CH)
    dst3 = jnp.concatenate([dst, pad_dst]).reshape(NW, KCH, CH)

    z2d64 = jnp.zeros((RPT, H), f32)
    z2d16 = jnp.zeros((RPT, 16), f32)
    z1d = jnp.zeros((RPT,), f32)

    y1 = _project_x(x, w1t)
    acc1, cnt = _sc_agg1(y1, src3, dst3, z2d64, z1d)
    y2 = _combine1(acc1, cnt, y1, b1r, w2t, b2r)
    acc2 = _sc_agg2(y2, src3, dst3, z2d16)
    return _combine2(acc2, cnt, y2)


# speedup vs baseline: 25.4384x; 1.0391x over previous
import functools

import jax
import jax.numpy as jnp
from jax import lax
from jax.experimental import pallas as pl
from jax.experimental.pallas import tpu as pltpu
from jax.experimental.pallas import tpu_sc as plsc

N = 10000
E = 160000
D_IN = 767
H = 64
D_OUT = 10

NPAD = 10240
NC = 2
NS = 16
NW = NC * NS
CH = 128
KCH = 40
NBUF1 = 4
NBUF2 = 8
TROW = N // NS
EPAD = NW * KCH * CH
RPT = NPAD // NS


def _mm_body(x_ref, w_ref, y_ref):
    y_ref[...] = jnp.dot(x_ref[...], w_ref[...],
                         preferred_element_type=jnp.float32)


def _project_x(x, w1t):
    return pl.pallas_call(
        _mm_body,
        grid=(5,),
        in_specs=[
            pl.BlockSpec((2000, D_IN), lambda i: (i, 0)),
            pl.BlockSpec((D_IN, 2 * H), lambda i: (0, 0)),
        ],
        out_specs=pl.BlockSpec((2000, 2 * H), lambda i: (i, 0)),
        out_shape=jax.ShapeDtypeStruct((N, 2 * H), jnp.float32),
    )(x, w1t)


def _combine1_body(acc_ref, cnt_ref, y1_ref, b1_ref, w2_ref, b2_ref, y2_ref):
    s = acc_ref[:N, :H] + acc_ref[:N, H:]
    c = cnt_ref[0, :N] + cnt_ref[1, :N]
    inv = 1.0 / jnp.maximum(c, 1.0)
    h = jnp.maximum(s * inv[:, None] + b1_ref[...] + y1_ref[:, H:], 0.0)
    y2_ref[...] = jnp.dot(h, w2_ref[...],
                          preferred_element_type=jnp.float32) + b2_ref[...]


def _combine1(acc1, cnt, y1, b1r, w2t, b2r):
    return pl.pallas_call(
        _combine1_body,
        out_shape=jax.ShapeDtypeStruct((N, 128), jnp.float32),
    )(acc1, cnt, y1, b1r, w2t, b2r)


def _combine2_body(acc_ref, cnt_ref, y2_ref, out_ref):
    s = acc_ref[:N, :16] + acc_ref[:N, 16:32]
    c = cnt_ref[0, :N] + cnt_ref[1, :N]
    inv = 1.0 / jnp.maximum(c, 1.0)
    out = s * inv[:, None] + y2_ref[:, 16:32]
    out_ref[...] = out[:, :D_OUT]


def _combine2(acc2, cnt, y2):
    return pl.pallas_call(
        _combine2_body,
        out_shape=jax.ShapeDtypeStruct((N, D_OUT), jnp.float32),
    )(acc2, cnt, y2)


def _sc_agg1(p1, src3, dst3, z2d, z1d):
    mesh = plsc.VectorSubcoreMesh(core_axis_name="c", subcore_axis_name="s")

    @functools.partial(
        pl.kernel,
        mesh=mesh,
        compiler_params=pltpu.CompilerParams(use_tc_tiling_on_sc=False),
        out_type=[
            jax.ShapeDtypeStruct((NPAD, 2 * H), jnp.float32),
            jax.ShapeDtypeStruct((NC, NPAD), jnp.float32),
        ],
        scratch_types=[
            pltpu.VMEM((KCH, CH), jnp.int32),
            pltpu.VMEM((KCH, CH), jnp.int32),
            pltpu.VMEM((NBUF1, CH, H), jnp.float32),
            pltpu.VMEM((CH,), jnp.float32),
            pltpu.VMEM_SHARED((N, H), jnp.float32),
            pltpu.VMEM_SHARED((NPAD, H), jnp.float32),
            pltpu.VMEM_SHARED((NPAD,), jnp.float32),
            [pltpu.SemaphoreType.DMA] * NBUF1,
            [pltpu.SemaphoreType.DMA] * NBUF1,
            pltpu.SemaphoreType.DMA,
        ],
    )
    def k(p1_hbm, src_hbm, dst_hbm, z2d_hbm, z1d_hbm, acc_out, cnt_out,
          src_v, dst_v, rows_v, ones_v, tbl_sh, acc_sh, cnt_sh,
          sems, ssems, osem):
        c = lax.axis_index("c")
        s = lax.axis_index("s")
        wid = s * NC + c
        base = s * RPT

        pltpu.sync_copy(src_hbm.at[wid], src_v)
        pltpu.sync_copy(dst_hbm.at[wid], dst_v)
        pltpu.sync_copy(p1_hbm.at[pl.ds(s * TROW, TROW), pl.ds(0, H)],
                        tbl_sh.at[pl.ds(s * TROW, TROW)])
        pltpu.sync_copy(z2d_hbm, acc_sh.at[pl.ds(base, RPT)])
        pltpu.sync_copy(z1d_hbm, cnt_sh.at[pl.ds(base, RPT)])
        for kk in range(CH // 16):
            ones_v[pl.ds(16 * kk, 16)] = jnp.full((16,), 1.0, jnp.float32)
        plsc.subcore_barrier()

        for b in range(NBUF1 - 1):
            pltpu.async_copy(tbl_sh.at[src_v.at[b]], rows_v.at[b], sems[b])

        def body(g, carry):
            for b in range(NBUF1):
                j = g * NBUF1 + b
                pltpu.make_async_copy(
                    tbl_sh.at[src_v.at[j]], rows_v.at[b], sems[b]).wait()
                pltpu.async_copy(
                    rows_v.at[b], acc_sh.at[dst_v.at[j]], ssems[b], add=True)
                pltpu.async_copy(
                    ones_v, cnt_sh.at[dst_v.at[j]], osem, add=True)

                bn = (b + NBUF1 - 1) % NBUF1

                @pl.when(j + NBUF1 - 1 < KCH)
                def _():
                    @pl.when(j >= 1)
                    def _():
                        pltpu.make_async_copy(
                            rows_v.at[bn], acc_sh.at[dst_v.at[0]],
                            ssems[bn]).wait()

                    pltpu.async_copy(
                        tbl_sh.at[src_v.at[j + NBUF1 - 1]], rows_v.at[bn],
                        sems[bn])
            return carry

        lax.fori_loop(0, KCH // NBUF1, body, 0)

        for b in range(NBUF1):
            pltpu.make_async_copy(
                rows_v.at[b], acc_sh.at[dst_v.at[0]], ssems[b]).wait()

        def drain(j, carry):
            pltpu.make_async_copy(
                ones_v, cnt_sh.at[dst_v.at[0]], osem).wait()
            return carry

        lax.fori_loop(0, KCH, drain, 0)
        plsc.subcore_barrier()

        pltpu.sync_copy(
            acc_sh.at[pl.ds(base, RPT)],
            acc_out.at[pl.ds(base, RPT), pl.ds(c * H, H)])
        pltpu.sync_copy(cnt_sh.at[pl.ds(base, RPT)],
                        cnt_out.at[c].at[pl.ds(base, RPT)])

    return k(p1, src3, dst3, z2d, z1d)


def _sc_agg2(p2, src3, dst3, z2d):
    mesh = plsc.VectorSubcoreMesh(core_axis_name="c", subcore_axis_name="s")

    @functools.partial(
        pl.kernel,
        mesh=mesh,
        compiler_params=pltpu.CompilerParams(use_tc_tiling_on_sc=False),
        out_type=jax.ShapeDtypeStruct((NPAD, 128), jnp.float32),
        scratch_types=[
            pltpu.VMEM((KCH, CH), jnp.int32),
            pltpu.VMEM((KCH, CH), jnp.int32),
            pltpu.VMEM((NBUF2, CH, 16), jnp.float32),
            pltpu.VMEM_SHARED((N, 16), jnp.float32),
            pltpu.VMEM_SHARED((NPAD, 16), jnp.float32),
            [pltpu.SemaphoreType.DMA] * NBUF2,
            [pltpu.SemaphoreType.DMA] * NBUF2,
        ],
    )
    def k(p2_hbm, src_hbm, dst_hbm, z2d_hbm, acc_out,
          src_v, dst_v, rows_v, tbl_sh, acc_sh, sems, ssems):
        c = lax.axis_index("c")
        s = lax.axis_index("s")
        wid = s * NC + c
        base = s * RPT

        pltpu.sync_copy(src_hbm.at[wid], src_v)
        pltpu.sync_copy(dst_hbm.at[wid], dst_v)
        pltpu.sync_copy(
            p2_hbm.at[pl.ds(s * TROW, TROW), pl.ds(0, 16)],
            tbl_sh.at[pl.ds(s * TROW, TROW)])
        pltpu.sync_copy(z2d_hbm, acc_sh.at[pl.ds(base, RPT)])
        plsc.subcore_barrier()

        for b in range(NBUF2 - 1):
            pltpu.async_copy(tbl_sh.at[src_v.at[b]], rows_v.at[b], sems[b])

        def body(g, carry):
            for b in range(NBUF2):
                j = g * NBUF2 + b
                pltpu.make_async_copy(
                    tbl_sh.at[src_v.at[j]], rows_v.at[b], sems[b]).wait()
                pltpu.async_copy(
                    rows_v.at[b], acc_sh.at[dst_v.at[j]], ssems[b], add=True)

                bn = (b + NBUF2 - 1) % NBUF2

                @pl.when(j + NBUF2 - 1 < KCH)
                def _():
                    @pl.when(j >= 1)
                    def _():
                        pltpu.make_async_copy(
                            rows_v.at[bn], acc_sh.at[dst_v.at[0]],
                            ssems[bn]).wait()

                    pltpu.async_copy(
                        tbl_sh.at[src_v.at[j + NBUF2 - 1]], rows_v.at[bn],
                        sems[bn])
            return carry

        lax.fori_loop(0, KCH // NBUF2, body, 0)

        for b in range(NBUF2):
            pltpu.make_async_copy(
                rows_v.at[b], acc_sh.at[dst_v.at[0]], ssems[b]).wait()

        plsc.subcore_barrier()

        pltpu.sync_copy(acc_sh.at[pl.ds(base, RPT)],
                        acc_out.at[pl.ds(base, RPT), pl.ds(c * 16, 16)])

    return k(p2, src3, dst3, z2d)


def kernel(x, edge_index, W1_l, b1_l, W1_r, W2_l, b2_l, W2_r):
    f32 = jnp.float32
    w1t = jnp.concatenate([W1_l, W1_r], axis=0).T
    w2t = jnp.zeros((H, 128), f32)
    w2t = w2t.at[:, :D_OUT].set(W2_l.T).at[:, 16:16 + D_OUT].set(W2_r.T)
    b1r = b1_l.reshape(1, H)
    b2r = jnp.zeros((1, 128), f32).at[0, 16:16 + D_OUT].set(b2_l)

    src = edge_index[0]
    dst = edge_index[1]
    pad_e = EPAD - E
    pad_dst = N + (jnp.arange(pad_e, dtype=jnp.int32) % (NPAD - N))
    src3 = jnp.concatenate(
        [src, jnp.zeros((pad_e,), jnp.int32)]).reshape(NW, KCH, CH)
    dst3 = jnp.concatenate([dst, pad_dst]).reshape(NW, KCH, CH)

    z2d64 = jnp.zeros((RPT, H), f32)
    z2d16 = jnp.zeros((RPT, 16), f32)
    z1d = jnp.zeros((RPT,), f32)

    y1 = _project_x(x, w1t)
    acc1, cnt = _sc_agg1(y1, src3, dst3, z2d64, z1d)
    y2 = _combine1(acc1, cnt, y1, b1r, w2t, b2r)
    acc2 = _sc_agg2(y2, src3, dst3, z2d16)
    return _combine2(acc2, cnt, y2)
